# Initial kernel scaffold; baseline (speedup 1.0000x reference)
import jax
import jax.numpy as jnp
from jax.experimental import pallas as pl


def kernel(x, edge_index, edge_attr, batch, params):
    raise NotImplementedError("write your pallas kernel here")


# speedup vs baseline: 1.8540x; 1.8540x over previous
import functools

import jax
import jax.numpy as jnp
from jax import lax
from jax.experimental import pallas as pl
from jax.experimental.pallas import tpu as pltpu
from jax.experimental.pallas import tpu_sc as plsc

N = 10000
E = 160000
H = 32
NW = 32
ROWS_PER_TILE = N // 16

_F32 = jnp.float32


def _sc_mesh():
    return plsc.VectorSubcoreMesh(core_axis_name="c", subcore_axis_name="s")


@functools.lru_cache(maxsize=None)
def _make_gather(n_rows, ch):
    wpw = n_rows // NW
    nch = wpw // ch

    @functools.partial(
        pl.kernel,
        out_type=jax.ShapeDtypeStruct((n_rows, H), _F32),
        mesh=_sc_mesh(),
        scratch_types=[
            pltpu.VMEM((ch,), jnp.int32),
            pltpu.VMEM((ch, H), _F32),
            pltpu.SemaphoreType.DMA,
        ],
        compiler_params=pltpu.CompilerParams(use_tc_tiling_on_sc=False),
    )
    def gather(table_hbm, idx_hbm, out_hbm, idx_v, rows_v, sem):
        wid = lax.axis_index("s") * 2 + lax.axis_index("c")
        base = wid * wpw

        def chunk(ci, carry):
            off = base + ci * ch
            pltpu.sync_copy(idx_hbm.at[pl.ds(off, ch)], idx_v)
            pltpu.async_copy(table_hbm.at[idx_v], rows_v, sem).wait()
            pltpu.sync_copy(rows_v, out_hbm.at[pl.ds(off, ch)])
            return carry

        lax.fori_loop(0, nch, chunk, 0, unroll=False)

    return gather


def _sc_gather_rows(table, idx):
    return _make_gather(idx.shape[0], 1000)(table, idx)


@functools.lru_cache(maxsize=None)
def _make_scatter(r, ch):
    wpw = E // NW
    nch = wpw // ch

    @functools.partial(
        pl.kernel,
        out_type=jax.ShapeDtypeStruct((2, N, r), _F32),
        mesh=_sc_mesh(),
        scratch_types=[
            pltpu.VMEM((ch,), jnp.int32),
            pltpu.VMEM((ch, r), _F32),
            pltpu.VMEM_SHARED((N, r), _F32),
        ],
        compiler_params=pltpu.CompilerParams(use_tc_tiling_on_sc=False),
    )
    def scatter(vals_hbm, idx_hbm, zeros_hbm, out_hbm, idx_v, vals_v, acc):
        c = lax.axis_index("c")
        s = lax.axis_index("s")
        wid = s * 2 + c
        r0 = s * ROWS_PER_TILE
        pltpu.sync_copy(zeros_hbm.at[pl.ds(r0, ROWS_PER_TILE)],
                        acc.at[pl.ds(r0, ROWS_PER_TILE)])
        plsc.subcore_barrier()

        base = wid * wpw

        def chunk(ci, carry):
            off = base + ci * ch
            pltpu.sync_copy(idx_hbm.at[pl.ds(off, ch)], idx_v)
            pltpu.sync_copy(vals_hbm.at[pl.ds(off, ch)], vals_v)
            pltpu.sync_copy(vals_v, acc.at[idx_v], add=True)
            return carry

        lax.fori_loop(0, nch, chunk, 0, unroll=False)
        plsc.subcore_barrier()
        pltpu.sync_copy(acc.at[pl.ds(r0, ROWS_PER_TILE)],
                        out_hbm.at[c, pl.ds(r0, ROWS_PER_TILE)])

    return scatter


def _sc_scatter_add(vals, idx, zeros, r):
    return _make_scatter(r, 1000)(vals, idx, zeros)


def _ln(v, g, b, eps=1e-5):
    mu = jnp.mean(v, axis=-1, keepdims=True)
    var = jnp.mean((v - mu) ** 2, axis=-1, keepdims=True)
    return (v - mu) / jnp.sqrt(var + eps) * g + b


def _gelu(v):
    return 0.5 * v * (1.0 + lax.erf(v / jnp.sqrt(jnp.float32(2.0))))


def _dot(a, b):
    return jnp.dot(a, b, preferred_element_type=_F32)


def _tc_edge_encode(edge_attr, w, b):
    eb = 2000

    def body(ea_ref, w_ref, b_ref, o_ref):
        o_ref[...] = _dot(ea_ref[...], w_ref[...]) + b_ref[...]

    return pl.pallas_call(
        body,
        grid=(E // eb,),
        in_specs=[
            pl.BlockSpec((eb, 16), lambda i: (i, 0)),
            pl.BlockSpec((16, H), lambda i: (0, 0)),
            pl.BlockSpec((1, H), lambda i: (0, 0)),
        ],
        out_specs=pl.BlockSpec((eb, H), lambda i: (i, 0)),
        out_shape=jax.ShapeDtypeStruct((E, H), _F32),
    )(edge_attr, w, b.reshape(1, H))


def _tc_node_pre(x, w_node, wl, bl, wr, br):
    def body(x_ref, wn_ref, wl_ref, bl_ref, wr_ref, br_ref, h_ref, lr_ref):
        h = _dot(x_ref[...], wn_ref[...])
        h_ref[...] = h
        lr_ref[:N] = _dot(h, wl_ref[...]) + bl_ref[...]
        lr_ref[N:] = _dot(h, wr_ref[...]) + br_ref[...]

    return pl.pallas_call(
        body,
        out_shape=[
            jax.ShapeDtypeStruct((N, H), _F32),
            jax.ShapeDtypeStruct((2 * N, H), _F32),
        ],
    )(x, w_node, wl, bl.reshape(1, H), wr, br.reshape(1, H))


def _tc_edge_gat(g, ea, we, att):
    eb = 2000
    nb = E // eb

    def body(xls_ref, xrd_ref, ea_ref, we_ref, att_ref, o_ref):
        xls = xls_ref[...]
        e = xls + xrd_ref[...] + _dot(ea_ref[...], we_ref[...])
        e = jnp.where(e > 0, e, 0.1 * e)
        s = jnp.sum(e * att_ref[...], axis=-1, keepdims=True)
        ex = jnp.exp(s)
        o_ref[...] = jnp.concatenate(
            [xls * ex, ex, jnp.zeros((eb, 15), _F32)], axis=1)

    return pl.pallas_call(
        body,
        grid=(nb,),
        in_specs=[
            pl.BlockSpec((eb, H), lambda i: (i, 0)),
            pl.BlockSpec((eb, H), lambda i: (i + nb, 0)),
            pl.BlockSpec((eb, H), lambda i: (i, 0)),
            pl.BlockSpec((H, H), lambda i: (0, 0)),
            pl.BlockSpec((1, H), lambda i: (0, 0)),
        ],
        out_specs=pl.BlockSpec((eb, 48), lambda i: (i, 0)),
        out_shape=jax.ShapeDtypeStruct((E, 48), _F32),
    )(g, g, ea, we, att.reshape(1, H))


def _tc_node_gat(parts, h, ln_g, ln_b, bias, nxt):
    def body(p_ref, h_ref, g_ref, b_ref, bias_ref, *rest):
        if nxt is None:
            (h_out,) = rest
        else:
            wl_ref, bl_ref, wr_ref, br_ref, h_out, lr_ref = rest
        u = p_ref[0] + p_ref[1]
        att_out = u[:, :H] / (u[:, H:H + 1] + 1e-16) + bias_ref[...]
        h1 = _gelu(_ln(h_ref[...] + att_out, g_ref[...], b_ref[...]))
        h_out[...] = h1
        if nxt is not None:
            lr_ref[:N] = _dot(h1, wl_ref[...]) + bl_ref[...]
            lr_ref[N:] = _dot(h1, wr_ref[...]) + br_ref[...]

    out_shape = [jax.ShapeDtypeStruct((N, H), _F32)]
    args = [parts, h, ln_g.reshape(1, H), ln_b.reshape(1, H),
            bias.reshape(1, H)]
    if nxt is not None:
        wl, bl, wr, br = nxt
        args += [wl, bl.reshape(1, H), wr, br.reshape(1, H)]
        out_shape.append(jax.ShapeDtypeStruct((2 * N, H), _F32))
    return pl.pallas_call(body, out_shape=out_shape)(*args)


def _tc_edge_nn(ea, xs, w2, b2):
    eb = 1000

    def body(ea_ref, xs_ref, w2_ref, b2_ref, o_ref):
        ea_v = ea_ref[...]
        xs_v = xs_ref[...]
        z = jnp.concatenate(
            [xs_v * ea_v[:, k:k + 1] for k in range(H)], axis=1)
        o_ref[...] = _dot(z, w2_ref[...]) + _dot(xs_v, b2_ref[...])

    return pl.pallas_call(
        body,
        grid=(E // eb,),
        in_specs=[
            pl.BlockSpec((eb, H), lambda i: (i, 0)),
            pl.BlockSpec((eb, H), lambda i: (i, 0)),
            pl.BlockSpec((H * H, H), lambda i: (0, 0)),
            pl.BlockSpec((H, H), lambda i: (0, 0)),
        ],
        out_specs=pl.BlockSpec((eb, H), lambda i: (i, 0)),
        out_shape=jax.ShapeDtypeStruct((E, H), _F32),
    )(ea, xs, w2, b2)


def _tc_node_dec(parts, xin, root, bias, x0, z_prev, ln_g, ln_b):
    def body(p_ref, xin_ref, root_ref, bias_ref, x0_ref, *rest):
        if z_prev is None:
            g_ref, b_ref, t_ref, z_ref = rest
            zp = 0.0
        else:
            zp_ref, g_ref, b_ref, t_ref, z_ref = rest
            zp = zp_ref[...]
        agg = p_ref[0] + p_ref[1]
        hd = zp + agg + _dot(xin_ref[...], root_ref[...]) + bias_ref[...]
        z = x0_ref[...] + hd
        z_ref[...] = z
        t_ref[...] = _gelu(_ln(z, g_ref[...], b_ref[...]))

    args = [parts, xin, root, bias.reshape(1, H), x0]
    if z_prev is not None:
        args.append(z_prev)
    args += [ln_g.reshape(1, H), ln_b.reshape(1, H)]
    return pl.pallas_call(
        body,
        out_shape=[jax.ShapeDtypeStruct((N, H), _F32),
                   jax.ShapeDtypeStruct((N, H), _F32)],
    )(*args)


def _tc_node_final(parts, xin, root, bias, z_prev, ln_g, ln_b, w_out, b_out):
    def body(p_ref, xin_ref, root_ref, bias_ref, zp_ref, g_ref, b_ref,
             wo_ref, bo_ref, o_ref):
        agg = p_ref[0] + p_ref[1]
        hd = (zp_ref[...] + agg + _dot(xin_ref[...], root_ref[...])
              + bias_ref[...])
        t = _gelu(_ln(hd, g_ref[...], b_ref[...]))
        o_ref[...] = _dot(t, wo_ref[...]) + bo_ref[...]

    return pl.pallas_call(
        body,
        out_shape=jax.ShapeDtypeStruct((N, 2), _F32),
    )(parts, xin, root, bias.reshape(1, H), z_prev, ln_g.reshape(1, H),
      ln_b.reshape(1, H), w_out, b_out.reshape(1, 2))


def kernel(x, edge_index, edge_attr, batch, params):
    src = edge_index[0]
    dst = edge_index[1]
    idx_lr = jnp.concatenate([src, dst + N])
    zeros48 = jnp.zeros((N, 48), _F32)
    zeros32 = jnp.zeros((N, H), _F32)

    enc = params['enc']
    dec = params['dec']

    ea = _tc_edge_encode(edge_attr, params['W_edge'], params['b_edge'])
    h, lr = _tc_node_pre(x, params['W_node'], enc[0]['Wl'], enc[0]['bl'],
                         enc[0]['Wr'], enc[0]['br'])

    for l in range(len(enc)):
        p = enc[l]
        g = _sc_gather_rows(lr, idx_lr)
        vals = _tc_edge_gat(g, ea, p['We'], p['att'])
        parts = _sc_scatter_add(vals, dst, zeros48, 48)
        if l + 1 < len(enc):
            q = enc[l + 1]
            h, lr = _tc_node_gat(parts, h, p['ln_g'], p['ln_b'], p['bias'],
                                 (q['Wl'], q['bl'], q['Wr'], q['br']))
        else:
            (h,) = _tc_node_gat(parts, h, p['ln_g'], p['ln_b'], p['bias'],
                                None)

    x0 = h
    w2 = [d['nn_W'].reshape(H * H, H) for d in dec]
    b2 = [d['nn_b'].reshape(H, H) for d in dec]

    xin, z = x0, None
    conv_params = [dec[0], dec[0], dec[1], dec[2]]
    conv_w = [(w2[0], b2[0]), (w2[0], b2[0]), (w2[1], b2[1]), (w2[2], b2[2])]
    ln_next = [dec[0], dec[1], dec[2]]

    for j in range(4):
        cp = conv_params[j]
        cw2, cb2 = conv_w[j]
        xs = _sc_gather_rows(xin, src)
        msg = _tc_edge_nn(ea, xs, cw2, cb2)
        parts = _sc_scatter_add(msg, dst, zeros32, H)
        if j < 3:
            nl = ln_next[j]
            t, z = _tc_node_dec(parts, xin, cp['root'], cp['bias'], x0, z,
                                nl['ln_g'], nl['ln_b'])
            xin = t
        else:
            out = _tc_node_final(parts, xin, cp['root'], cp['bias'], z,
                                 dec[0]['ln_g'], dec[0]['ln_b'],
                                 params['W_out'], params['b_out'])
    return out


# speedup vs baseline: 4.2109x; 2.2712x over previous
import functools

import jax
import jax.numpy as jnp
from jax import lax
from jax.experimental import pallas as pl
from jax.experimental.pallas import tpu as pltpu
from jax.experimental.pallas import tpu_sc as plsc

N = 10000
E = 160000
H = 32
NW = 32
ROWS_PER_TILE = N // 16

_F32 = jnp.float32


def _sc_mesh():
    return plsc.VectorSubcoreMesh(core_axis_name="c", subcore_axis_name="s")


@functools.lru_cache(maxsize=None)
def _make_gather(n_rows, ch):
    wpw = n_rows // NW
    nch = wpw // ch

    @functools.partial(
        pl.kernel,
        out_type=jax.ShapeDtypeStruct((n_rows, H), _F32),
        mesh=_sc_mesh(),
        scratch_types=[
            pltpu.VMEM((ch,), jnp.int32),
            pltpu.VMEM((ch, H), _F32),
            pltpu.SemaphoreType.DMA,
        ],
        compiler_params=pltpu.CompilerParams(use_tc_tiling_on_sc=False),
    )
    def gather(table_hbm, idx_hbm, out_hbm, idx_v, rows_v, sem):
        wid = lax.axis_index("s") * 2 + lax.axis_index("c")
        base = wid * wpw

        def chunk(ci, carry):
            off = base + ci * ch
            pltpu.sync_copy(idx_hbm.at[pl.ds(off, ch)], idx_v)
            pltpu.async_copy(table_hbm.at[idx_v], rows_v, sem).wait()
            pltpu.sync_copy(rows_v, out_hbm.at[pl.ds(off, ch)])
            return carry

        lax.fori_loop(0, nch, chunk, 0, unroll=False)

    return gather


def _sc_gather_rows(table, idx):
    return _make_gather(idx.shape[0], 1000)(table, idx)


@functools.lru_cache(maxsize=None)
def _make_scatter(r, ch):
    wpw = E // NW
    nch = wpw // ch

    @functools.partial(
        pl.kernel,
        out_type=jax.ShapeDtypeStruct((2, N, r), _F32),
        mesh=_sc_mesh(),
        scratch_types=[
            pltpu.VMEM((ch,), jnp.int32),
            pltpu.VMEM((ch, r), _F32),
            pltpu.VMEM_SHARED((N, r), _F32),
        ],
        compiler_params=pltpu.CompilerParams(use_tc_tiling_on_sc=False),
    )
    def scatter(vals_hbm, idx_hbm, zeros_hbm, out_hbm, idx_v, vals_v, acc):
        c = lax.axis_index("c")
        s = lax.axis_index("s")
        wid = s * 2 + c
        r0 = s * ROWS_PER_TILE
        pltpu.sync_copy(zeros_hbm.at[pl.ds(r0, ROWS_PER_TILE)],
                        acc.at[pl.ds(r0, ROWS_PER_TILE)])
        plsc.subcore_barrier()

        base = wid * wpw

        def chunk(ci, carry):
            off = base + ci * ch
            pltpu.sync_copy(idx_hbm.at[pl.ds(off, ch)], idx_v)
            pltpu.sync_copy(vals_hbm.at[pl.ds(off, ch)], vals_v)
            pltpu.sync_copy(vals_v, acc.at[idx_v], add=True)
            return carry

        lax.fori_loop(0, nch, chunk, 0, unroll=False)
        plsc.subcore_barrier()
        pltpu.sync_copy(acc.at[pl.ds(r0, ROWS_PER_TILE)],
                        out_hbm.at[c, pl.ds(r0, ROWS_PER_TILE)])

    return scatter


def _sc_scatter_add(vals, idx, zeros, r):
    return _make_scatter(r, 1000)(vals, idx, zeros)


def _ln(v, g, b, eps=1e-5):
    mu = jnp.mean(v, axis=-1, keepdims=True)
    var = jnp.mean((v - mu) ** 2, axis=-1, keepdims=True)
    return (v - mu) / jnp.sqrt(var + eps) * g + b


def _gelu(v):
    return 0.5 * v * (1.0 + lax.erf(v / jnp.sqrt(jnp.float32(2.0))))


def _dot(a, b):
    return jnp.dot(a, b, preferred_element_type=_F32)


def _tc_edge_encode(edge_attr, w, b):
    eb = 2000

    def body(ea_ref, w_ref, b_ref, o_ref):
        o_ref[...] = _dot(ea_ref[...], w_ref[...]) + b_ref[...]

    return pl.pallas_call(
        body,
        grid=(E // eb,),
        in_specs=[
            pl.BlockSpec((eb, 16), lambda i: (i, 0)),
            pl.BlockSpec((16, H), lambda i: (0, 0)),
            pl.BlockSpec((1, H), lambda i: (0, 0)),
        ],
        out_specs=pl.BlockSpec((eb, H), lambda i: (i, 0)),
        out_shape=jax.ShapeDtypeStruct((E, H), _F32),
    )(edge_attr, w, b.reshape(1, H))


def _tc_node_pre(x, w_node, wl, bl, wr, br):
    def body(x_ref, wn_ref, wl_ref, bl_ref, wr_ref, br_ref, h_ref, lr_ref):
        h = _dot(x_ref[...], wn_ref[...])
        h_ref[...] = h
        lr_ref[:N] = _dot(h, wl_ref[...]) + bl_ref[...]
        lr_ref[N:] = _dot(h, wr_ref[...]) + br_ref[...]

    return pl.pallas_call(
        body,
        out_shape=[
            jax.ShapeDtypeStruct((N, H), _F32),
            jax.ShapeDtypeStruct((2 * N, H), _F32),
        ],
    )(x, w_node, wl, bl.reshape(1, H), wr, br.reshape(1, H))


def _tc_edge_gat(g, ea, we, att):
    eb = 2000
    nb = E // eb

    def body(xls_ref, xrd_ref, ea_ref, we_ref, att_ref, o_ref):
        xls = xls_ref[...]
        e = xls + xrd_ref[...] + _dot(ea_ref[...], we_ref[...])
        e = jnp.where(e > 0, e, 0.1 * e)
        s = jnp.sum(e * att_ref[...], axis=-1, keepdims=True)
        ex = jnp.exp(s)
        o_ref[...] = jnp.concatenate(
            [xls * ex, ex, jnp.zeros((eb, 15), _F32)], axis=1)

    return pl.pallas_call(
        body,
        grid=(nb,),
        in_specs=[
            pl.BlockSpec((eb, H), lambda i: (i, 0)),
            pl.BlockSpec((eb, H), lambda i: (i + nb, 0)),
            pl.BlockSpec((eb, H), lambda i: (i, 0)),
            pl.BlockSpec((H, H), lambda i: (0, 0)),
            pl.BlockSpec((1, H), lambda i: (0, 0)),
        ],
        out_specs=pl.BlockSpec((eb, 48), lambda i: (i, 0)),
        out_shape=jax.ShapeDtypeStruct((E, 48), _F32),
    )(g, g, ea, we, att.reshape(1, H))


def _tc_node_gat(parts, h, ln_g, ln_b, bias, nxt):
    def body(p_ref, h_ref, g_ref, b_ref, bias_ref, *rest):
        if nxt is None:
            (h_out,) = rest
        else:
            wl_ref, bl_ref, wr_ref, br_ref, h_out, lr_ref = rest
        u = p_ref[0] + p_ref[1]
        att_out = u[:, :H] / (u[:, H:H + 1] + 1e-16) + bias_ref[...]
        h1 = _gelu(_ln(h_ref[...] + att_out, g_ref[...], b_ref[...]))
        h_out[...] = h1
        if nxt is not None:
            lr_ref[:N] = _dot(h1, wl_ref[...]) + bl_ref[...]
            lr_ref[N:] = _dot(h1, wr_ref[...]) + br_ref[...]

    out_shape = [jax.ShapeDtypeStruct((N, H), _F32)]
    args = [parts, h, ln_g.reshape(1, H), ln_b.reshape(1, H),
            bias.reshape(1, H)]
    if nxt is not None:
        wl, bl, wr, br = nxt
        args += [wl, bl.reshape(1, H), wr, br.reshape(1, H)]
        out_shape.append(jax.ShapeDtypeStruct((2 * N, H), _F32))
    return pl.pallas_call(body, out_shape=out_shape)(*args)


def _tc_edge_nn(ea, xs, w2ext):
    eb = 1000

    def body(ea_ref, xs_ref, w2_ref, o_ref):
        ident = jax.lax.broadcasted_iota(jnp.int32, (H, H), 0) == \
            jax.lax.broadcasted_iota(jnp.int32, (H, H), 1)
        ident = ident.astype(_F32)
        eat = jax.lax.dot_general(ident, ea_ref[...],
                                  (((1,), (1,)), ((), ())),
                                  preferred_element_type=_F32)
        xst = jax.lax.dot_general(ident, xs_ref[...],
                                  (((1,), (1,)), ((), ())),
                                  preferred_element_type=_F32)
        zt = (eat[:, None, :] * xst[None, :, :]).reshape(H * H, eb)
        zt = jnp.concatenate([zt, xst], axis=0)
        o_ref[...] = jax.lax.dot_general(zt, w2_ref[...],
                                         (((0,), (0,)), ((), ())),
                                         preferred_element_type=_F32)

    return pl.pallas_call(
        body,
        grid=(E // eb,),
        in_specs=[
            pl.BlockSpec((eb, H), lambda i: (i, 0)),
            pl.BlockSpec((eb, H), lambda i: (i, 0)),
            pl.BlockSpec((H * H + H, H), lambda i: (0, 0)),
        ],
        out_specs=pl.BlockSpec((eb, H), lambda i: (i, 0)),
        out_shape=jax.ShapeDtypeStruct((E, H), _F32),
    )(ea, xs, w2ext)


def _tc_node_dec(parts, xin, root, bias, x0, z_prev, ln_g, ln_b):
    def body(p_ref, xin_ref, root_ref, bias_ref, x0_ref, *rest):
        if z_prev is None:
            g_ref, b_ref, t_ref, z_ref = rest
            zp = 0.0
        else:
            zp_ref, g_ref, b_ref, t_ref, z_ref = rest
            zp = zp_ref[...]
        agg = p_ref[0] + p_ref[1]
        hd = zp + agg + _dot(xin_ref[...], root_ref[...]) + bias_ref[...]
        z = x0_ref[...] + hd
        z_ref[...] = z
        t_ref[...] = _gelu(_ln(z, g_ref[...], b_ref[...]))

    args = [parts, xin, root, bias.reshape(1, H), x0]
    if z_prev is not None:
        args.append(z_prev)
    args += [ln_g.reshape(1, H), ln_b.reshape(1, H)]
    return pl.pallas_call(
        body,
        out_shape=[jax.ShapeDtypeStruct((N, H), _F32),
                   jax.ShapeDtypeStruct((N, H), _F32)],
    )(*args)


def _tc_node_final(parts, xin, root, bias, z_prev, ln_g, ln_b, w_out, b_out):
    def body(p_ref, xin_ref, root_ref, bias_ref, zp_ref, g_ref, b_ref,
             wo_ref, bo_ref, o_ref):
        agg = p_ref[0] + p_ref[1]
        hd = (zp_ref[...] + agg + _dot(xin_ref[...], root_ref[...])
              + bias_ref[...])
        t = _gelu(_ln(hd, g_ref[...], b_ref[...]))
        o_ref[...] = _dot(t, wo_ref[...]) + bo_ref[...]

    return pl.pallas_call(
        body,
        out_shape=jax.ShapeDtypeStruct((N, 2), _F32),
    )(parts, xin, root, bias.reshape(1, H), z_prev, ln_g.reshape(1, H),
      ln_b.reshape(1, H), w_out, b_out.reshape(1, 2))


def kernel(x, edge_index, edge_attr, batch, params):
    src = edge_index[0]
    dst = edge_index[1]
    idx_lr = jnp.concatenate([src, dst + N])
    zeros48 = jnp.zeros((N, 48), _F32)
    zeros32 = jnp.zeros((N, H), _F32)

    enc = params['enc']
    dec = params['dec']

    ea = _tc_edge_encode(edge_attr, params['W_edge'], params['b_edge'])
    h, lr = _tc_node_pre(x, params['W_node'], enc[0]['Wl'], enc[0]['bl'],
                         enc[0]['Wr'], enc[0]['br'])

    for l in range(len(enc)):
        p = enc[l]
        g = _sc_gather_rows(lr, idx_lr)
        vals = _tc_edge_gat(g, ea, p['We'], p['att'])
        parts = _sc_scatter_add(vals, dst, zeros48, 48)
        if l + 1 < len(enc):
            q = enc[l + 1]
            h, lr = _tc_node_gat(parts, h, p['ln_g'], p['ln_b'], p['bias'],
                                 (q['Wl'], q['bl'], q['Wr'], q['br']))
        else:
            (h,) = _tc_node_gat(parts, h, p['ln_g'], p['ln_b'], p['bias'],
                                None)

    x0 = h
    w2 = [jnp.concatenate([d['nn_W'].reshape(H * H, H),
                           d['nn_b'].reshape(H, H)], axis=0) for d in dec]

    xin, z = x0, None
    conv_params = [dec[0], dec[0], dec[1], dec[2]]
    conv_w = [w2[0], w2[0], w2[1], w2[2]]
    ln_next = [dec[0], dec[1], dec[2]]

    for j in range(4):
        cp = conv_params[j]
        xs = _sc_gather_rows(xin, src)
        msg = _tc_edge_nn(ea, xs, conv_w[j])
        parts = _sc_scatter_add(msg, dst, zeros32, H)
        if j < 3:
            nl = ln_next[j]
            t, z = _tc_node_dec(parts, xin, cp['root'], cp['bias'], x0, z,
                                nl['ln_g'], nl['ln_b'])
            xin = t
        else:
            out = _tc_node_final(parts, xin, cp['root'], cp['bias'], z,
                                 dec[0]['ln_g'], dec[0]['ln_b'],
                                 params['W_out'], params['b_out'])
    return out


# speedup vs baseline: 4.3161x; 1.0250x over previous
import functools

import jax
import jax.numpy as jnp
from jax import lax
from jax.experimental import pallas as pl
from jax.experimental.pallas import tpu as pltpu
from jax.experimental.pallas import tpu_sc as plsc

N = 10000
E = 160000
H = 32
NW = 32
ROWS_PER_TILE = N // 16

_F32 = jnp.float32


def _sc_mesh():
    return plsc.VectorSubcoreMesh(core_axis_name="c", subcore_axis_name="s")


_GCH = 200
_GNB = 5


@functools.lru_cache(maxsize=None)
def _make_gather(n_rows):
    wpw = n_rows // NW
    nch = wpw // _GCH
    ngrp = nch // _GNB

    @functools.partial(
        pl.kernel,
        out_type=jax.ShapeDtypeStruct((n_rows, H), _F32),
        mesh=_sc_mesh(),
        scratch_types=[
            pltpu.VMEM((wpw,), jnp.int32),
            pltpu.VMEM((_GNB, _GCH, H), _F32),
            pltpu.SemaphoreType.DMA,
            pltpu.SemaphoreType.DMA((_GNB,)),
            pltpu.SemaphoreType.DMA((_GNB,)),
        ],
        compiler_params=pltpu.CompilerParams(use_tc_tiling_on_sc=False),
    )
    def gather(table_hbm, idx_hbm, out_hbm, idx_all, rows, isem, gsem, wsem):
        wid = lax.axis_index("s") * 2 + lax.axis_index("c")
        base = wid * wpw
        pltpu.async_copy(idx_hbm.at[pl.ds(base, wpw)], idx_all, isem).wait()

        def group(gi, carry):
            cps = []
            for b in range(_GNB):
                off = (gi * _GNB + b) * _GCH

                @pl.when(gi > 0)
                def _wait_prev_write(b=b):
                    pltpu.make_async_copy(
                        rows.at[b], out_hbm.at[pl.ds(base, _GCH)],
                        wsem.at[b]).wait()

                cps.append(pltpu.async_copy(
                    table_hbm.at[idx_all.at[pl.ds(off, _GCH)]],
                    rows.at[b], gsem.at[b]))
            for b in range(_GNB):
                off = (gi * _GNB + b) * _GCH
                cps[b].wait()
                pltpu.async_copy(rows.at[b],
                                 out_hbm.at[pl.ds(base + off, _GCH)],
                                 wsem.at[b])
            return carry

        lax.fori_loop(0, ngrp, group, 0, unroll=False)
        for b in range(_GNB):
            pltpu.make_async_copy(rows.at[b], out_hbm.at[pl.ds(base, _GCH)],
                                  wsem.at[b]).wait()

    return gather


def _sc_gather_rows(table, idx):
    return _make_gather(idx.shape[0])(table, idx)


@functools.lru_cache(maxsize=None)
def _make_scatter(r):
    wpw = E // NW
    nch = wpw // _GCH
    ngrp = nch // _GNB

    @functools.partial(
        pl.kernel,
        out_type=jax.ShapeDtypeStruct((2, N, r), _F32),
        mesh=_sc_mesh(),
        scratch_types=[
            pltpu.VMEM((_GNB, _GCH), jnp.int32),
            pltpu.VMEM((_GNB, _GCH, r), _F32),
            pltpu.VMEM_SHARED((N, r), _F32),
            pltpu.SemaphoreType.DMA((_GNB,)),
            pltpu.SemaphoreType.DMA((_GNB,)),
            pltpu.SemaphoreType.DMA((_GNB,)),
        ],
        compiler_params=pltpu.CompilerParams(use_tc_tiling_on_sc=False),
    )
    def scatter(vals_hbm, idx_hbm, zeros_hbm, out_hbm, idx_v, vals_v, acc,
                isem, lsem, ssem):
        c = lax.axis_index("c")
        s = lax.axis_index("s")
        wid = s * 2 + c
        r0 = s * ROWS_PER_TILE
        pltpu.sync_copy(zeros_hbm.at[pl.ds(r0, ROWS_PER_TILE)],
                        acc.at[pl.ds(r0, ROWS_PER_TILE)])
        plsc.subcore_barrier()

        base = wid * wpw

        def group(gi, carry):
            icps, vcps = [], []
            for b in range(_GNB):
                off = base + (gi * _GNB + b) * _GCH

                @pl.when(gi > 0)
                def _wait_prev_scatter(b=b):
                    pltpu.make_async_copy(
                        vals_v.at[b], acc.at[idx_v.at[b]], ssem.at[b]).wait()

                icps.append(pltpu.async_copy(idx_hbm.at[pl.ds(off, _GCH)],
                                             idx_v.at[b], isem.at[b]))
                vcps.append(pltpu.async_copy(vals_hbm.at[pl.ds(off, _GCH)],
                                             vals_v.at[b], lsem.at[b]))
            for b in range(_GNB):
                icps[b].wait()
                vcps[b].wait()
                pltpu.async_copy(vals_v.at[b], acc.at[idx_v.at[b]],
                                 ssem.at[b], add=True)
            return carry

        lax.fori_loop(0, ngrp, group, 0, unroll=False)
        for b in range(_GNB):
            pltpu.make_async_copy(vals_v.at[b], acc.at[idx_v.at[b]],
                                  ssem.at[b]).wait()
        plsc.subcore_barrier()
        pltpu.sync_copy(acc.at[pl.ds(r0, ROWS_PER_TILE)],
                        out_hbm.at[c, pl.ds(r0, ROWS_PER_TILE)])

    return scatter


def _sc_scatter_add(vals, idx, zeros, r):
    return _make_scatter(r)(vals, idx, zeros)


def _ln(v, g, b, eps=1e-5):
    mu = jnp.mean(v, axis=-1, keepdims=True)
    var = jnp.mean((v - mu) ** 2, axis=-1, keepdims=True)
    return (v - mu) / jnp.sqrt(var + eps) * g + b


def _gelu(v):
    return 0.5 * v * (1.0 + lax.erf(v / jnp.sqrt(jnp.float32(2.0))))


def _dot(a, b):
    return jnp.dot(a, b, preferred_element_type=_F32)


def _tc_edge_encode(edge_attr, w, b):
    eb = 2000

    def body(ea_ref, w_ref, b_ref, o_ref):
        o_ref[...] = _dot(ea_ref[...], w_ref[...]) + b_ref[...]

    return pl.pallas_call(
        body,
        grid=(E // eb,),
        in_specs=[
            pl.BlockSpec((eb, 16), lambda i: (i, 0)),
            pl.BlockSpec((16, H), lambda i: (0, 0)),
            pl.BlockSpec((1, H), lambda i: (0, 0)),
        ],
        out_specs=pl.BlockSpec((eb, H), lambda i: (i, 0)),
        out_shape=jax.ShapeDtypeStruct((E, H), _F32),
    )(edge_attr, w, b.reshape(1, H))


def _tc_node_pre(x, w_node, wl, bl, wr, br):
    def body(x_ref, wn_ref, wl_ref, bl_ref, wr_ref, br_ref, h_ref, lr_ref):
        h = _dot(x_ref[...], wn_ref[...])
        h_ref[...] = h
        lr_ref[:N] = _dot(h, wl_ref[...]) + bl_ref[...]
        lr_ref[N:] = _dot(h, wr_ref[...]) + br_ref[...]

    return pl.pallas_call(
        body,
        out_shape=[
            jax.ShapeDtypeStruct((N, H), _F32),
            jax.ShapeDtypeStruct((2 * N, H), _F32),
        ],
    )(x, w_node, wl, bl.reshape(1, H), wr, br.reshape(1, H))


def _tc_edge_gat(g, ea, we, att):
    eb = 2000
    nb = E // eb

    def body(xls_ref, xrd_ref, ea_ref, we_ref, att_ref, o_ref):
        xls = xls_ref[...]
        e = xls + xrd_ref[...] + _dot(ea_ref[...], we_ref[...])
        e = jnp.where(e > 0, e, 0.1 * e)
        s = jnp.sum(e * att_ref[...], axis=-1, keepdims=True)
        ex = jnp.exp(s)
        o_ref[...] = jnp.concatenate(
            [xls * ex, ex, jnp.zeros((eb, 15), _F32)], axis=1)

    return pl.pallas_call(
        body,
        grid=(nb,),
        in_specs=[
            pl.BlockSpec((eb, H), lambda i: (i, 0)),
            pl.BlockSpec((eb, H), lambda i: (i + nb, 0)),
            pl.BlockSpec((eb, H), lambda i: (i, 0)),
            pl.BlockSpec((H, H), lambda i: (0, 0)),
            pl.BlockSpec((1, H), lambda i: (0, 0)),
        ],
        out_specs=pl.BlockSpec((eb, 48), lambda i: (i, 0)),
        out_shape=jax.ShapeDtypeStruct((E, 48), _F32),
    )(g, g, ea, we, att.reshape(1, H))


def _tc_node_gat(parts, h, ln_g, ln_b, bias, nxt):
    def body(p_ref, h_ref, g_ref, b_ref, bias_ref, *rest):
        if nxt is None:
            (h_out,) = rest
        else:
            wl_ref, bl_ref, wr_ref, br_ref, h_out, lr_ref = rest
        u = p_ref[0] + p_ref[1]
        att_out = u[:, :H] / (u[:, H:H + 1] + 1e-16) + bias_ref[...]
        h1 = _gelu(_ln(h_ref[...] + att_out, g_ref[...], b_ref[...]))
        h_out[...] = h1
        if nxt is not None:
            lr_ref[:N] = _dot(h1, wl_ref[...]) + bl_ref[...]
            lr_ref[N:] = _dot(h1, wr_ref[...]) + br_ref[...]

    out_shape = [jax.ShapeDtypeStruct((N, H), _F32)]
    args = [parts, h, ln_g.reshape(1, H), ln_b.reshape(1, H),
            bias.reshape(1, H)]
    if nxt is not None:
        wl, bl, wr, br = nxt
        args += [wl, bl.reshape(1, H), wr, br.reshape(1, H)]
        out_shape.append(jax.ShapeDtypeStruct((2 * N, H), _F32))
    return pl.pallas_call(body, out_shape=out_shape)(*args)


def _tc_edge_nn(ea, xs, w2ext):
    eb = 1000

    def body(ea_ref, xs_ref, w2_ref, o_ref):
        ident = jax.lax.broadcasted_iota(jnp.int32, (H, H), 0) == \
            jax.lax.broadcasted_iota(jnp.int32, (H, H), 1)
        ident = ident.astype(_F32)
        eat = jax.lax.dot_general(ident, ea_ref[...],
                                  (((1,), (1,)), ((), ())),
                                  preferred_element_type=_F32)
        xst = jax.lax.dot_general(ident, xs_ref[...],
                                  (((1,), (1,)), ((), ())),
                                  preferred_element_type=_F32)
        zt = (eat[:, None, :] * xst[None, :, :]).reshape(H * H, eb)
        zt = jnp.concatenate([zt, xst], axis=0)
        o_ref[...] = jax.lax.dot_general(zt, w2_ref[...],
                                         (((0,), (0,)), ((), ())),
                                         preferred_element_type=_F32)

    return pl.pallas_call(
        body,
        grid=(E // eb,),
        in_specs=[
            pl.BlockSpec((eb, H), lambda i: (i, 0)),
            pl.BlockSpec((eb, H), lambda i: (i, 0)),
            pl.BlockSpec((H * H + H, H), lambda i: (0, 0)),
        ],
        out_specs=pl.BlockSpec((eb, H), lambda i: (i, 0)),
        out_shape=jax.ShapeDtypeStruct((E, H), _F32),
    )(ea, xs, w2ext)


def _tc_node_dec(parts, xin, root, bias, x0, z_prev, ln_g, ln_b):
    def body(p_ref, xin_ref, root_ref, bias_ref, x0_ref, *rest):
        if z_prev is None:
            g_ref, b_ref, t_ref, z_ref = rest
            zp = 0.0
        else:
            zp_ref, g_ref, b_ref, t_ref, z_ref = rest
            zp = zp_ref[...]
        agg = p_ref[0] + p_ref[1]
        hd = zp + agg + _dot(xin_ref[...], root_ref[...]) + bias_ref[...]
        z = x0_ref[...] + hd
        z_ref[...] = z
        t_ref[...] = _gelu(_ln(z, g_ref[...], b_ref[...]))

    args = [parts, xin, root, bias.reshape(1, H), x0]
    if z_prev is not None:
        args.append(z_prev)
    args += [ln_g.reshape(1, H), ln_b.reshape(1, H)]
    return pl.pallas_call(
        body,
        out_shape=[jax.ShapeDtypeStruct((N, H), _F32),
                   jax.ShapeDtypeStruct((N, H), _F32)],
    )(*args)


def _tc_node_final(parts, xin, root, bias, z_prev, ln_g, ln_b, w_out, b_out):
    def body(p_ref, xin_ref, root_ref, bias_ref, zp_ref, g_ref, b_ref,
             wo_ref, bo_ref, o_ref):
        agg = p_ref[0] + p_ref[1]
        hd = (zp_ref[...] + agg + _dot(xin_ref[...], root_ref[...])
              + bias_ref[...])
        t = _gelu(_ln(hd, g_ref[...], b_ref[...]))
        o_ref[...] = _dot(t, wo_ref[...]) + bo_ref[...]

    return pl.pallas_call(
        body,
        out_shape=jax.ShapeDtypeStruct((N, 2), _F32),
    )(parts, xin, root, bias.reshape(1, H), z_prev, ln_g.reshape(1, H),
      ln_b.reshape(1, H), w_out, b_out.reshape(1, 2))


def kernel(x, edge_index, edge_attr, batch, params):
    src = edge_index[0]
    dst = edge_index[1]
    idx_lr = jnp.concatenate([src, dst + N])
    zeros48 = jnp.zeros((N, 48), _F32)
    zeros32 = jnp.zeros((N, H), _F32)

    enc = params['enc']
    dec = params['dec']

    ea = _tc_edge_encode(edge_attr, params['W_edge'], params['b_edge'])
    h, lr = _tc_node_pre(x, params['W_node'], enc[0]['Wl'], enc[0]['bl'],
                         enc[0]['Wr'], enc[0]['br'])

    for l in range(len(enc)):
        p = enc[l]
        g = _sc_gather_rows(lr, idx_lr)
        vals = _tc_edge_gat(g, ea, p['We'], p['att'])
        parts = _sc_scatter_add(vals, dst, zeros48, 48)
        if l + 1 < len(enc):
            q = enc[l + 1]
            h, lr = _tc_node_gat(parts, h, p['ln_g'], p['ln_b'], p['bias'],
                                 (q['Wl'], q['bl'], q['Wr'], q['br']))
        else:
            (h,) = _tc_node_gat(parts, h, p['ln_g'], p['ln_b'], p['bias'],
                                None)

    x0 = h
    w2 = [jnp.concatenate([d['nn_W'].reshape(H * H, H),
                           d['nn_b'].reshape(H, H)], axis=0) for d in dec]

    xin, z = x0, None
    conv_params = [dec[0], dec[0], dec[1], dec[2]]
    conv_w = [w2[0], w2[0], w2[1], w2[2]]
    ln_next = [dec[0], dec[1], dec[2]]

    for j in range(4):
        cp = conv_params[j]
        xs = _sc_gather_rows(xin, src)
        msg = _tc_edge_nn(ea, xs, conv_w[j])
        parts = _sc_scatter_add(msg, dst, zeros32, H)
        if j < 3:
            nl = ln_next[j]
            t, z = _tc_node_dec(parts, xin, cp['root'], cp['bias'], x0, z,
                                nl['ln_g'], nl['ln_b'])
            xin = t
        else:
            out = _tc_node_final(parts, xin, cp['root'], cp['bias'], z,
                                 dec[0]['ln_g'], dec[0]['ln_b'],
                                 params['W_out'], params['b_out'])
    return out


# speedup vs baseline: 4.8467x; 1.1229x over previous
import functools

import jax
import jax.numpy as jnp
from jax import lax
from jax.experimental import pallas as pl
from jax.experimental.pallas import tpu as pltpu
from jax.experimental.pallas import tpu_sc as plsc

N = 10000
E = 160000
H = 32
W = 128
NW = 32
ROWS_PER_TILE = N // 16

_GCH = 40
_GNB = 5

_F32 = jnp.float32


def _sc_mesh():
    return plsc.VectorSubcoreMesh(core_axis_name="c", subcore_axis_name="s")


@functools.lru_cache(maxsize=None)
def _make_gather(n_rows):
    wpw = n_rows // NW
    nch = wpw // _GCH
    ngrp = nch // _GNB

    @functools.partial(
        pl.kernel,
        out_type=jax.ShapeDtypeStruct((n_rows, W), _F32),
        mesh=_sc_mesh(),
        scratch_types=[
            pltpu.VMEM((wpw,), jnp.int32),
            pltpu.VMEM((_GNB, _GCH, W), _F32),
            pltpu.SemaphoreType.DMA,
            pltpu.SemaphoreType.DMA((_GNB,)),
            pltpu.SemaphoreType.DMA((_GNB,)),
        ],
        compiler_params=pltpu.CompilerParams(use_tc_tiling_on_sc=False),
    )
    def gather(table_hbm, idx_hbm, out_hbm, idx_all, rows, isem, gsem, wsem):
        wid = lax.axis_index("s") * 2 + lax.axis_index("c")
        base = wid * wpw
        pltpu.async_copy(idx_hbm.at[pl.ds(base, wpw)], idx_all, isem).wait()

        def group(gi, carry):
            cps = []
            for b in range(_GNB):
                off = (gi * _GNB + b) * _GCH

                @pl.when(gi > 0)
                def _wait_prev_write(b=b):
                    pltpu.make_async_copy(
                        rows.at[b], out_hbm.at[pl.ds(base, _GCH)],
                        wsem.at[b]).wait()

                cps.append(pltpu.async_copy(
                    table_hbm.at[idx_all.at[pl.ds(off, _GCH)]],
                    rows.at[b], gsem.at[b]))
            for b in range(_GNB):
                off = (gi * _GNB + b) * _GCH
                cps[b].wait()
                pltpu.async_copy(rows.at[b],
                                 out_hbm.at[pl.ds(base + off, _GCH)],
                                 wsem.at[b])
            return carry

        lax.fori_loop(0, ngrp, group, 0, unroll=False)
        for b in range(_GNB):
            pltpu.make_async_copy(rows.at[b], out_hbm.at[pl.ds(base, _GCH)],
                                  wsem.at[b]).wait()

    return gather


def _sc_gather_rows(table, idx):
    return _make_gather(idx.shape[0])(table, idx)


@functools.lru_cache(maxsize=None)
def _make_scatter():
    wpw = E // NW
    nch = wpw // _GCH
    ngrp = nch // _GNB

    @functools.partial(
        pl.kernel,
        out_type=jax.ShapeDtypeStruct((2, N, W), _F32),
        mesh=_sc_mesh(),
        scratch_types=[
            pltpu.VMEM((_GNB, _GCH), jnp.int32),
            pltpu.VMEM((_GNB, _GCH, W), _F32),
            pltpu.VMEM_SHARED((N, W), _F32),
            pltpu.SemaphoreType.DMA((_GNB,)),
            pltpu.SemaphoreType.DMA((_GNB,)),
            pltpu.SemaphoreType.DMA((_GNB,)),
        ],
        compiler_params=pltpu.CompilerParams(use_tc_tiling_on_sc=False),
    )
    def scatter(vals_hbm, idx_hbm, zeros_hbm, out_hbm, idx_v, vals_v, acc,
                isem, lsem, ssem):
        c = lax.axis_index("c")
        s = lax.axis_index("s")
        wid = s * 2 + c
        r0 = s * ROWS_PER_TILE
        pltpu.sync_copy(zeros_hbm, acc.at[pl.ds(r0, ROWS_PER_TILE)])
        plsc.subcore_barrier()

        base = wid * wpw

        def group(gi, carry):
            icps, vcps = [], []
            for b in range(_GNB):
                off = base + (gi * _GNB + b) * _GCH

                @pl.when(gi > 0)
                def _wait_prev_scatter(b=b):
                    pltpu.make_async_copy(
                        vals_v.at[b], acc.at[idx_v.at[b]], ssem.at[b]).wait()

                icps.append(pltpu.async_copy(idx_hbm.at[pl.ds(off, _GCH)],
                                             idx_v.at[b], isem.at[b]))
                vcps.append(pltpu.async_copy(vals_hbm.at[pl.ds(off, _GCH)],
                                             vals_v.at[b], lsem.at[b]))
            for b in range(_GNB):
                icps[b].wait()
                vcps[b].wait()
                pltpu.async_copy(vals_v.at[b], acc.at[idx_v.at[b]],
                                 ssem.at[b], add=True)
            return carry

        lax.fori_loop(0, ngrp, group, 0, unroll=False)
        for b in range(_GNB):
            pltpu.make_async_copy(vals_v.at[b], acc.at[idx_v.at[b]],
                                  ssem.at[b]).wait()
        plsc.subcore_barrier()
        pltpu.sync_copy(acc.at[pl.ds(r0, ROWS_PER_TILE)],
                        out_hbm.at[c, pl.ds(r0, ROWS_PER_TILE)])

    return scatter


def _sc_scatter_add(vals, idx, zeros):
    return _make_scatter()(vals, idx, zeros)


def _ln(v, g, b, eps=1e-5):
    mu = jnp.mean(v, axis=-1, keepdims=True)
    var = jnp.mean((v - mu) ** 2, axis=-1, keepdims=True)
    return (v - mu) / jnp.sqrt(var + eps) * g + b


def _gelu(v):
    return 0.5 * v * (1.0 + lax.erf(v / jnp.sqrt(jnp.float32(2.0))))


def _dot(a, b):
    return jnp.dot(a, b, preferred_element_type=_F32)


def _padw(v, rows):
    return jnp.concatenate([v, jnp.zeros((rows, W - v.shape[1]), _F32)],
                           axis=1)


def _tc_edge_encode(edge_attr, w, b):
    eb = 2000

    def body(ea_ref, w_ref, b_ref, o_ref):
        o_ref[...] = _dot(ea_ref[...], w_ref[...]) + b_ref[...]

    return pl.pallas_call(
        body,
        grid=(E // eb,),
        in_specs=[
            pl.BlockSpec((eb, 16), lambda i: (i, 0)),
            pl.BlockSpec((16, H), lambda i: (0, 0)),
            pl.BlockSpec((1, H), lambda i: (0, 0)),
        ],
        out_specs=pl.BlockSpec((eb, H), lambda i: (i, 0)),
        out_shape=jax.ShapeDtypeStruct((E, H), _F32),
    )(edge_attr, w, b.reshape(1, H))


def _tc_node_pre(x, w_node, wl, bl, wr, br):
    def body(x_ref, wn_ref, wl_ref, bl_ref, wr_ref, br_ref, h_ref, lr_ref):
        h = _dot(x_ref[...], wn_ref[...])
        h_ref[...] = h
        lr_ref[:N] = _padw(_dot(h, wl_ref[...]) + bl_ref[...], N)
        lr_ref[N:] = _padw(_dot(h, wr_ref[...]) + br_ref[...], N)

    return pl.pallas_call(
        body,
        out_shape=[
            jax.ShapeDtypeStruct((N, H), _F32),
            jax.ShapeDtypeStruct((2 * N, W), _F32),
        ],
    )(x, w_node, wl, bl.reshape(1, H), wr, br.reshape(1, H))


def _tc_edge_gat(g, ea, we, att):
    eb = 2000
    nb = E // eb

    def body(xls_ref, xrd_ref, ea_ref, we_ref, att_ref, o_ref):
        xls = xls_ref[:, :H]
        e = xls + xrd_ref[:, :H] + _dot(ea_ref[...], we_ref[...])
        e = jnp.where(e > 0, e, 0.1 * e)
        s = jnp.sum(e * att_ref[...], axis=-1, keepdims=True)
        ex = jnp.exp(s)
        o_ref[...] = _padw(jnp.concatenate([xls * ex, ex], axis=1), eb)

    return pl.pallas_call(
        body,
        grid=(nb,),
        in_specs=[
            pl.BlockSpec((eb, W), lambda i: (i, 0)),
            pl.BlockSpec((eb, W), lambda i: (i + nb, 0)),
            pl.BlockSpec((eb, H), lambda i: (i, 0)),
            pl.BlockSpec((H, H), lambda i: (0, 0)),
            pl.BlockSpec((1, H), lambda i: (0, 0)),
        ],
        out_specs=pl.BlockSpec((eb, W), lambda i: (i, 0)),
        out_shape=jax.ShapeDtypeStruct((E, W), _F32),
    )(g, g, ea, we, att.reshape(1, H))


def _tc_node_gat(parts, h, ln_g, ln_b, bias, nxt, pad_h):
    def body(p_ref, h_ref, g_ref, b_ref, bias_ref, *rest):
        if nxt is None:
            (h_out,) = rest
        else:
            wl_ref, bl_ref, wr_ref, br_ref, h_out, lr_ref = rest
        u = p_ref[0] + p_ref[1]
        att_out = u[:, :H] / (u[:, H:H + 1] + 1e-16) + bias_ref[...]
        h1 = _gelu(_ln(h_ref[...] + att_out, g_ref[...], b_ref[...]))
        h_out[...] = _padw(h1, N) if pad_h else h1
        if nxt is not None:
            lr_ref[:N] = _padw(_dot(h1, wl_ref[...]) + bl_ref[...], N)
            lr_ref[N:] = _padw(_dot(h1, wr_ref[...]) + br_ref[...], N)

    out_shape = [jax.ShapeDtypeStruct((N, W if pad_h else H), _F32)]
    args = [parts, h, ln_g.reshape(1, H), ln_b.reshape(1, H),
            bias.reshape(1, H)]
    if nxt is not None:
        wl, bl, wr, br = nxt
        args += [wl, bl.reshape(1, H), wr, br.reshape(1, H)]
        out_shape.append(jax.ShapeDtypeStruct((2 * N, W), _F32))
    return pl.pallas_call(body, out_shape=out_shape)(*args)


def _tc_edge_nn(ea, xs, w2ext):
    eb = 1000

    def body(ea_ref, xs_ref, w2_ref, o_ref):
        ident = jax.lax.broadcasted_iota(jnp.int32, (H, H), 0) == \
            jax.lax.broadcasted_iota(jnp.int32, (H, H), 1)
        ident = ident.astype(_F32)
        eat = jax.lax.dot_general(ident, ea_ref[...],
                                  (((1,), (1,)), ((), ())),
                                  preferred_element_type=_F32)
        xst = jax.lax.dot_general(ident, xs_ref[:, :H],
                                  (((1,), (1,)), ((), ())),
                                  preferred_element_type=_F32)
        zt = (eat[:, None, :] * xst[None, :, :]).reshape(H * H, eb)
        zt = jnp.concatenate([zt, xst], axis=0)
        msg = jax.lax.dot_general(zt, w2_ref[...],
                                  (((0,), (0,)), ((), ())),
                                  preferred_element_type=_F32)
        o_ref[...] = _padw(msg, eb)

    return pl.pallas_call(
        body,
        grid=(E // eb,),
        in_specs=[
            pl.BlockSpec((eb, H), lambda i: (i, 0)),
            pl.BlockSpec((eb, W), lambda i: (i, 0)),
            pl.BlockSpec((H * H + H, H), lambda i: (0, 0)),
        ],
        out_specs=pl.BlockSpec((eb, W), lambda i: (i, 0)),
        out_shape=jax.ShapeDtypeStruct((E, W), _F32),
    )(ea, xs, w2ext)


def _tc_node_dec(parts, xin, root, bias, x0, z_prev, ln_g, ln_b):
    def body(p_ref, xin_ref, root_ref, bias_ref, x0_ref, *rest):
        if z_prev is None:
            g_ref, b_ref, t_ref, z_ref = rest
            zp = 0.0
        else:
            zp_ref, g_ref, b_ref, t_ref, z_ref = rest
            zp = zp_ref[...]
        agg = p_ref[0][:, :H] + p_ref[1][:, :H]
        hd = zp + agg + _dot(xin_ref[:, :H], root_ref[...]) + bias_ref[...]
        z = x0_ref[...] + hd
        z_ref[...] = z
        t_ref[...] = _padw(_gelu(_ln(z, g_ref[...], b_ref[...])), N)

    args = [parts, xin, root, bias.reshape(1, H), x0]
    if z_prev is not None:
        args.append(z_prev)
    args += [ln_g.reshape(1, H), ln_b.reshape(1, H)]
    return pl.pallas_call(
        body,
        out_shape=[jax.ShapeDtypeStruct((N, W), _F32),
                   jax.ShapeDtypeStruct((N, H), _F32)],
    )(*args)


def _tc_node_final(parts, xin, root, bias, z_prev, ln_g, ln_b, w_out, b_out):
    def body(p_ref, xin_ref, root_ref, bias_ref, zp_ref, g_ref, b_ref,
             wo_ref, bo_ref, o_ref):
        agg = p_ref[0][:, :H] + p_ref[1][:, :H]
        hd = (zp_ref[...] + agg + _dot(xin_ref[:, :H], root_ref[...])
              + bias_ref[...])
        t = _gelu(_ln(hd, g_ref[...], b_ref[...]))
        o_ref[...] = _dot(t, wo_ref[...]) + bo_ref[...]

    return pl.pallas_call(
        body,
        out_shape=jax.ShapeDtypeStruct((N, 2), _F32),
    )(parts, xin, root, bias.reshape(1, H), z_prev, ln_g.reshape(1, H),
      ln_b.reshape(1, H), w_out, b_out.reshape(1, 2))


def kernel(x, edge_index, edge_attr, batch, params):
    src = edge_index[0]
    dst = edge_index[1]
    idx_lr = jnp.concatenate([src, dst + N])
    zeros = jnp.zeros((ROWS_PER_TILE, W), _F32)

    enc = params['enc']
    dec = params['dec']

    ea = _tc_edge_encode(edge_attr, params['W_edge'], params['b_edge'])
    h, lr = _tc_node_pre(x, params['W_node'], enc[0]['Wl'], enc[0]['bl'],
                         enc[0]['Wr'], enc[0]['br'])

    for l in range(len(enc)):
        p = enc[l]
        g = _sc_gather_rows(lr, idx_lr)
        vals = _tc_edge_gat(g, ea, p['We'], p['att'])
        parts = _sc_scatter_add(vals, dst, zeros)
        if l + 1 < len(enc):
            q = enc[l + 1]
            h, lr = _tc_node_gat(parts, h, p['ln_g'], p['ln_b'], p['bias'],
                                 (q['Wl'], q['bl'], q['Wr'], q['br']), False)
        else:
            (h,) = _tc_node_gat(parts, h, p['ln_g'], p['ln_b'], p['bias'],
                                None, True)

    xpad = h
    x0 = None
    w2 = [jnp.concatenate([d['nn_W'].reshape(H * H, H),
                           d['nn_b'].reshape(H, H)], axis=0) for d in dec]

    conv_params = [dec[0], dec[0], dec[1], dec[2]]
    conv_w = [w2[0], w2[0], w2[1], w2[2]]
    ln_next = [dec[0], dec[1], dec[2]]

    x0_narrow = xpad[:, :H]
    xin, z = xpad, None
    for j in range(4):
        cp = conv_params[j]
        xs = _sc_gather_rows(xin, src)
        msg = _tc_edge_nn(ea, xs, conv_w[j])
        parts = _sc_scatter_add(msg, dst, zeros)
        if j < 3:
            nl = ln_next[j]
            t, z = _tc_node_dec(parts, xin, cp['root'], cp['bias'],
                                x0_narrow, z, nl['ln_g'], nl['ln_b'])
            xin = t
        else:
            out = _tc_node_final(parts, xin, cp['root'], cp['bias'], z,
                                 dec[0]['ln_g'], dec[0]['ln_b'],
                                 params['W_out'], params['b_out'])
    return out


# speedup vs baseline: 4.8543x; 1.0016x over previous
import functools

import jax
import jax.numpy as jnp
from jax import lax
from jax.experimental import pallas as pl
from jax.experimental.pallas import tpu as pltpu
from jax.experimental.pallas import tpu_sc as plsc

N = 10000
E = 160000
H = 32
W = 128
NW = 32
ROWS_PER_TILE = N // 16

_GCH = 40
_GNB = 5

_F32 = jnp.float32


def _sc_mesh():
    return plsc.VectorSubcoreMesh(core_axis_name="c", subcore_axis_name="s")


@functools.lru_cache(maxsize=None)
def _make_gather(n_rows, w):
    ch = 40 if w == W else 200
    wpw = n_rows // NW
    nch = wpw // ch
    ngrp = nch // _GNB

    @functools.partial(
        pl.kernel,
        out_type=jax.ShapeDtypeStruct((n_rows, w), _F32),
        mesh=_sc_mesh(),
        scratch_types=[
            pltpu.VMEM((wpw,), jnp.int32),
            pltpu.VMEM((_GNB, ch, w), _F32),
            pltpu.SemaphoreType.DMA,
            pltpu.SemaphoreType.DMA((_GNB,)),
            pltpu.SemaphoreType.DMA((_GNB,)),
        ],
        compiler_params=pltpu.CompilerParams(use_tc_tiling_on_sc=False),
    )
    def gather(table_hbm, idx_hbm, out_hbm, idx_all, rows, isem, gsem, wsem):
        wid = lax.axis_index("s") * 2 + lax.axis_index("c")
        base = wid * wpw
        pltpu.async_copy(idx_hbm.at[pl.ds(base, wpw)], idx_all, isem).wait()

        def group(gi, carry):
            cps = []
            for b in range(_GNB):
                off = (gi * _GNB + b) * ch

                @pl.when(gi > 0)
                def _wait_prev_write(b=b):
                    pltpu.make_async_copy(
                        rows.at[b], out_hbm.at[pl.ds(base, ch)],
                        wsem.at[b]).wait()

                cps.append(pltpu.async_copy(
                    table_hbm.at[idx_all.at[pl.ds(off, ch)]],
                    rows.at[b], gsem.at[b]))
            for b in range(_GNB):
                off = (gi * _GNB + b) * ch
                cps[b].wait()
                pltpu.async_copy(rows.at[b],
                                 out_hbm.at[pl.ds(base + off, ch)],
                                 wsem.at[b])
            return carry

        lax.fori_loop(0, ngrp, group, 0, unroll=False)
        for b in range(_GNB):
            pltpu.make_async_copy(rows.at[b], out_hbm.at[pl.ds(base, ch)],
                                  wsem.at[b]).wait()

    return gather


def _sc_gather_rows(table, idx):
    return _make_gather(idx.shape[0], table.shape[1])(table, idx)


@functools.lru_cache(maxsize=None)
def _make_scatter(w):
    ch = 40 if w == W else 200
    wpw = E // NW
    nch = wpw // ch
    ngrp = nch // _GNB

    @functools.partial(
        pl.kernel,
        out_type=jax.ShapeDtypeStruct((2, N, w), _F32),
        mesh=_sc_mesh(),
        scratch_types=[
            pltpu.VMEM((_GNB, ch), jnp.int32),
            pltpu.VMEM((_GNB, ch, w), _F32),
            pltpu.VMEM_SHARED((N, w), _F32),
            pltpu.SemaphoreType.DMA((_GNB,)),
            pltpu.SemaphoreType.DMA((_GNB,)),
            pltpu.SemaphoreType.DMA((_GNB,)),
        ],
        compiler_params=pltpu.CompilerParams(use_tc_tiling_on_sc=False),
    )
    def scatter(vals_hbm, idx_hbm, zeros_hbm, out_hbm, idx_v, vals_v, acc,
                isem, lsem, ssem):
        c = lax.axis_index("c")
        s = lax.axis_index("s")
        wid = s * 2 + c
        r0 = s * ROWS_PER_TILE
        pltpu.sync_copy(zeros_hbm, acc.at[pl.ds(r0, ROWS_PER_TILE)])
        plsc.subcore_barrier()

        base = wid * wpw

        def group(gi, carry):
            icps, vcps = [], []
            for b in range(_GNB):
                off = base + (gi * _GNB + b) * ch

                @pl.when(gi > 0)
                def _wait_prev_scatter(b=b):
                    pltpu.make_async_copy(
                        vals_v.at[b], acc.at[idx_v.at[b]], ssem.at[b]).wait()

                icps.append(pltpu.async_copy(idx_hbm.at[pl.ds(off, ch)],
                                             idx_v.at[b], isem.at[b]))
                vcps.append(pltpu.async_copy(vals_hbm.at[pl.ds(off, ch)],
                                             vals_v.at[b], lsem.at[b]))
            for b in range(_GNB):
                icps[b].wait()
                vcps[b].wait()
                pltpu.async_copy(vals_v.at[b], acc.at[idx_v.at[b]],
                                 ssem.at[b], add=True)
            return carry

        lax.fori_loop(0, ngrp, group, 0, unroll=False)
        for b in range(_GNB):
            pltpu.make_async_copy(vals_v.at[b], acc.at[idx_v.at[b]],
                                  ssem.at[b]).wait()
        plsc.subcore_barrier()
        pltpu.sync_copy(acc.at[pl.ds(r0, ROWS_PER_TILE)],
                        out_hbm.at[c, pl.ds(r0, ROWS_PER_TILE)])

    return scatter


def _sc_scatter_add(vals, idx, zeros):
    return _make_scatter(vals.shape[1])(vals, idx, zeros)


def _ln(v, g, b, eps=1e-5):
    mu = jnp.mean(v, axis=-1, keepdims=True)
    var = jnp.mean((v - mu) ** 2, axis=-1, keepdims=True)
    return (v - mu) / jnp.sqrt(var + eps) * g + b


def _gelu(v):
    return 0.5 * v * (1.0 + lax.erf(v / jnp.sqrt(jnp.float32(2.0))))


def _dot(a, b):
    return jnp.dot(a, b, preferred_element_type=_F32)


def _padw(v, rows):
    return jnp.concatenate([v, jnp.zeros((rows, W - v.shape[1]), _F32)],
                           axis=1)


def _tc_edge_encode(edge_attr, w, b):
    eb = 2000

    def body(ea_ref, w_ref, b_ref, o_ref):
        o_ref[...] = _dot(ea_ref[...], w_ref[...]) + b_ref[...]

    return pl.pallas_call(
        body,
        grid=(E // eb,),
        in_specs=[
            pl.BlockSpec((eb, 16), lambda i: (i, 0)),
            pl.BlockSpec((16, H), lambda i: (0, 0)),
            pl.BlockSpec((1, H), lambda i: (0, 0)),
        ],
        out_specs=pl.BlockSpec((eb, H), lambda i: (i, 0)),
        out_shape=jax.ShapeDtypeStruct((E, H), _F32),
    )(edge_attr, w, b.reshape(1, H))


def _tc_node_pre(x, w_node, wl, bl, wr, br):
    def body(x_ref, wn_ref, wl_ref, bl_ref, wr_ref, br_ref, h_ref, lr_ref):
        h = _dot(x_ref[...], wn_ref[...])
        h_ref[...] = h
        lr_ref[:N] = _padw(_dot(h, wl_ref[...]) + bl_ref[...], N)
        lr_ref[N:] = _padw(_dot(h, wr_ref[...]) + br_ref[...], N)

    return pl.pallas_call(
        body,
        out_shape=[
            jax.ShapeDtypeStruct((N, H), _F32),
            jax.ShapeDtypeStruct((2 * N, W), _F32),
        ],
    )(x, w_node, wl, bl.reshape(1, H), wr, br.reshape(1, H))


def _tc_edge_gat(g, ea, we, att):
    eb = 2000
    nb = E // eb

    def body(xls_ref, xrd_ref, ea_ref, we_ref, att_ref, o_ref):
        xls = xls_ref[:, :H]
        e = xls + xrd_ref[:, :H] + _dot(ea_ref[...], we_ref[...])
        e = jnp.where(e > 0, e, 0.1 * e)
        s = jnp.sum(e * att_ref[...], axis=-1, keepdims=True)
        ex = jnp.exp(s)
        o_ref[...] = _padw(jnp.concatenate([xls * ex, ex], axis=1), eb)

    return pl.pallas_call(
        body,
        grid=(nb,),
        in_specs=[
            pl.BlockSpec((eb, W), lambda i: (i, 0)),
            pl.BlockSpec((eb, W), lambda i: (i + nb, 0)),
            pl.BlockSpec((eb, H), lambda i: (i, 0)),
            pl.BlockSpec((H, H), lambda i: (0, 0)),
            pl.BlockSpec((1, H), lambda i: (0, 0)),
        ],
        out_specs=pl.BlockSpec((eb, W), lambda i: (i, 0)),
        out_shape=jax.ShapeDtypeStruct((E, W), _F32),
    )(g, g, ea, we, att.reshape(1, H))


def _tc_node_gat(parts, h, ln_g, ln_b, bias, nxt):
    def body(p_ref, h_ref, g_ref, b_ref, bias_ref, *rest):
        if nxt is None:
            (h_out,) = rest
        else:
            wl_ref, bl_ref, wr_ref, br_ref, h_out, lr_ref = rest
        u = p_ref[0] + p_ref[1]
        att_out = u[:, :H] / (u[:, H:H + 1] + 1e-16) + bias_ref[...]
        h1 = _gelu(_ln(h_ref[...] + att_out, g_ref[...], b_ref[...]))
        h_out[...] = h1
        if nxt is not None:
            lr_ref[:N] = _padw(_dot(h1, wl_ref[...]) + bl_ref[...], N)
            lr_ref[N:] = _padw(_dot(h1, wr_ref[...]) + br_ref[...], N)

    out_shape = [jax.ShapeDtypeStruct((N, H), _F32)]
    args = [parts, h, ln_g.reshape(1, H), ln_b.reshape(1, H),
            bias.reshape(1, H)]
    if nxt is not None:
        wl, bl, wr, br = nxt
        args += [wl, bl.reshape(1, H), wr, br.reshape(1, H)]
        out_shape.append(jax.ShapeDtypeStruct((2 * N, W), _F32))
    return pl.pallas_call(body, out_shape=out_shape)(*args)


def _tc_edge_nn(ea_p, xs_p, w2ext):
    eb4 = 400

    def body(ea_ref, xs_ref, w2_ref, o_ref):
        ident = jax.lax.broadcasted_iota(jnp.int32, (H, H), 0) == \
            jax.lax.broadcasted_iota(jnp.int32, (H, H), 1)
        ident = ident.astype(_F32)
        msgs = []
        for g in range(4):
            ea_g = ea_ref[:, H * g:H * (g + 1)]
            xs_g = xs_ref[:, H * g:H * (g + 1)]
            eat = jax.lax.dot_general(ident, ea_g, (((1,), (1,)), ((), ())),
                                      preferred_element_type=_F32)
            xst = jax.lax.dot_general(ident, xs_g, (((1,), (1,)), ((), ())),
                                      preferred_element_type=_F32)
            zt = (eat[:, None, :] * xst[None, :, :]).reshape(H * H, eb4)
            zt = jnp.concatenate([zt, xst], axis=0)
            msgs.append(jax.lax.dot_general(zt, w2_ref[...],
                                            (((0,), (0,)), ((), ())),
                                            preferred_element_type=_F32))
        o_ref[...] = jnp.concatenate(msgs, axis=1)

    return pl.pallas_call(
        body,
        grid=(E // (4 * eb4),),
        in_specs=[
            pl.BlockSpec((eb4, W), lambda i: (i, 0)),
            pl.BlockSpec((eb4, W), lambda i: (i, 0)),
            pl.BlockSpec((H * H + H, H), lambda i: (0, 0)),
        ],
        out_specs=pl.BlockSpec((eb4, W), lambda i: (i, 0)),
        out_shape=jax.ShapeDtypeStruct((E // 4, W), _F32),
    )(ea_p, xs_p, w2ext)


def _tc_node_dec(parts, xin, root, bias, x0, z_prev, ln_g, ln_b):
    def body(p_ref, xin_ref, root_ref, bias_ref, x0_ref, *rest):
        if z_prev is None:
            g_ref, b_ref, t_ref, z_ref = rest
            zp = 0.0
        else:
            zp_ref, g_ref, b_ref, t_ref, z_ref = rest
            zp = zp_ref[...]
        agg = p_ref[0] + p_ref[1]
        hd = zp + agg + _dot(xin_ref[...], root_ref[...]) + bias_ref[...]
        z = x0_ref[...] + hd
        z_ref[...] = z
        t_ref[...] = _gelu(_ln(z, g_ref[...], b_ref[...]))

    args = [parts, xin, root, bias.reshape(1, H), x0]
    if z_prev is not None:
        args.append(z_prev)
    args += [ln_g.reshape(1, H), ln_b.reshape(1, H)]
    return pl.pallas_call(
        body,
        out_shape=[jax.ShapeDtypeStruct((N, H), _F32),
                   jax.ShapeDtypeStruct((N, H), _F32)],
    )(*args)


def _tc_node_final(parts, xin, root, bias, z_prev, ln_g, ln_b, w_out, b_out):
    def body(p_ref, xin_ref, root_ref, bias_ref, zp_ref, g_ref, b_ref,
             wo_ref, bo_ref, o_ref):
        agg = p_ref[0] + p_ref[1]
        hd = (zp_ref[...] + agg + _dot(xin_ref[...], root_ref[...])
              + bias_ref[...])
        t = _gelu(_ln(hd, g_ref[...], b_ref[...]))
        o_ref[...] = _dot(t, wo_ref[...]) + bo_ref[...]

    return pl.pallas_call(
        body,
        out_shape=jax.ShapeDtypeStruct((N, 2), _F32),
    )(parts, xin, root, bias.reshape(1, H), z_prev, ln_g.reshape(1, H),
      ln_b.reshape(1, H), w_out, b_out.reshape(1, 2))


def kernel(x, edge_index, edge_attr, batch, params):
    src = edge_index[0]
    dst = edge_index[1]
    idx_lr = jnp.concatenate([src, dst + N])
    zeros_w = jnp.zeros((ROWS_PER_TILE, W), _F32)
    zeros_h = jnp.zeros((ROWS_PER_TILE, H), _F32)

    enc = params['enc']
    dec = params['dec']

    ea = _tc_edge_encode(edge_attr, params['W_edge'], params['b_edge'])
    ea_p = jnp.reshape(ea, (E // 4, W))
    h, lr = _tc_node_pre(x, params['W_node'], enc[0]['Wl'], enc[0]['bl'],
                         enc[0]['Wr'], enc[0]['br'])

    for l in range(len(enc)):
        p = enc[l]
        g = _sc_gather_rows(lr, idx_lr)
        vals = _tc_edge_gat(g, ea, p['We'], p['att'])
        parts = _sc_scatter_add(vals, dst, zeros_w)
        if l + 1 < len(enc):
            q = enc[l + 1]
            h, lr = _tc_node_gat(parts, h, p['ln_g'], p['ln_b'], p['bias'],
                                 (q['Wl'], q['bl'], q['Wr'], q['br']))
        else:
            (h,) = _tc_node_gat(parts, h, p['ln_g'], p['ln_b'], p['bias'],
                                None)

    x0 = h
    w2 = [jnp.concatenate([d['nn_W'].reshape(H * H, H),
                           d['nn_b'].reshape(H, H)], axis=0) for d in dec]

    conv_params = [dec[0], dec[0], dec[1], dec[2]]
    conv_w = [w2[0], w2[0], w2[1], w2[2]]
    ln_next = [dec[0], dec[1], dec[2]]

    xin, z = x0, None
    for j in range(4):
        cp = conv_params[j]
        xs = _sc_gather_rows(xin, src)
        msg_p = _tc_edge_nn(ea_p, jnp.reshape(xs, (E // 4, W)), conv_w[j])
        parts = _sc_scatter_add(jnp.reshape(msg_p, (E, H)), dst, zeros_h)
        if j < 3:
            nl = ln_next[j]
            t, z = _tc_node_dec(parts, xin, cp['root'], cp['bias'],
                                x0, z, nl['ln_g'], nl['ln_b'])
            xin = t
        else:
            out = _tc_node_final(parts, xin, cp['root'], cp['bias'], z,
                                 dec[0]['ln_g'], dec[0]['ln_b'],
                                 params['W_out'], params['b_out'])
    return out


# speedup vs baseline: 6.8229x; 1.4055x over previous
import functools

import jax
import jax.numpy as jnp
from jax import lax
from jax.experimental import pallas as pl
from jax.experimental.pallas import tpu as pltpu
from jax.experimental.pallas import tpu_sc as plsc

N = 10000
E = 160000
H = 32
W = 128
NW = 32
ROWS_PER_TILE = N // 16

_GCH = 40
_GNB = 5

_F32 = jnp.float32


def _sc_mesh():
    return plsc.VectorSubcoreMesh(core_axis_name="c", subcore_axis_name="s")


@functools.lru_cache(maxsize=None)
def _make_gather(n_rows, w):
    ch = 40 if w == W else 200
    wpw = n_rows // NW
    nch = wpw // ch
    ngrp = nch // _GNB

    @functools.partial(
        pl.kernel,
        out_type=jax.ShapeDtypeStruct((n_rows, w), _F32),
        mesh=_sc_mesh(),
        scratch_types=[
            pltpu.VMEM((wpw,), jnp.int32),
            pltpu.VMEM((_GNB, ch, w), _F32),
            pltpu.SemaphoreType.DMA,
            pltpu.SemaphoreType.DMA((_GNB,)),
            pltpu.SemaphoreType.DMA((_GNB,)),
        ],
        compiler_params=pltpu.CompilerParams(use_tc_tiling_on_sc=False),
    )
    def gather(table_hbm, idx_hbm, out_hbm, idx_all, rows, isem, gsem, wsem):
        wid = lax.axis_index("s") * 2 + lax.axis_index("c")
        base = wid * wpw
        pltpu.async_copy(idx_hbm.at[pl.ds(base, wpw)], idx_all, isem).wait()

        def group(gi, carry):
            cps = []
            for b in range(_GNB):
                off = (gi * _GNB + b) * ch

                @pl.when(gi > 0)
                def _wait_prev_write(b=b):
                    pltpu.make_async_copy(
                        rows.at[b], out_hbm.at[pl.ds(base, ch)],
                        wsem.at[b]).wait()

                cps.append(pltpu.async_copy(
                    table_hbm.at[idx_all.at[pl.ds(off, ch)]],
                    rows.at[b], gsem.at[b]))
            for b in range(_GNB):
                off = (gi * _GNB + b) * ch
                cps[b].wait()
                pltpu.async_copy(rows.at[b],
                                 out_hbm.at[pl.ds(base + off, ch)],
                                 wsem.at[b])
            return carry

        lax.fori_loop(0, ngrp, group, 0, unroll=False)
        for b in range(_GNB):
            pltpu.make_async_copy(rows.at[b], out_hbm.at[pl.ds(base, ch)],
                                  wsem.at[b]).wait()

    return gather


def _sc_gather_rows(table, idx):
    return _make_gather(idx.shape[0], table.shape[1])(table, idx)


@functools.lru_cache(maxsize=None)
def _make_scatter(w):
    ch = 40 if w == W else 200
    wpw = E // NW
    nch = wpw // ch
    ngrp = nch // _GNB

    @functools.partial(
        pl.kernel,
        out_type=jax.ShapeDtypeStruct((2, N, w), _F32),
        mesh=_sc_mesh(),
        scratch_types=[
            pltpu.VMEM((_GNB, ch), jnp.int32),
            pltpu.VMEM((_GNB, ch, w), _F32),
            pltpu.VMEM_SHARED((N, w), _F32),
            pltpu.SemaphoreType.DMA((_GNB,)),
            pltpu.SemaphoreType.DMA((_GNB,)),
            pltpu.SemaphoreType.DMA((_GNB,)),
        ],
        compiler_params=pltpu.CompilerParams(use_tc_tiling_on_sc=False),
    )
    def scatter(vals_hbm, idx_hbm, zeros_hbm, out_hbm, idx_v, vals_v, acc,
                isem, lsem, ssem):
        c = lax.axis_index("c")
        s = lax.axis_index("s")
        wid = s * 2 + c
        r0 = s * ROWS_PER_TILE
        pltpu.sync_copy(zeros_hbm, acc.at[pl.ds(r0, ROWS_PER_TILE)])
        plsc.subcore_barrier()

        base = wid * wpw

        def group(gi, carry):
            icps, vcps = [], []
            for b in range(_GNB):
                off = base + (gi * _GNB + b) * ch

                @pl.when(gi > 0)
                def _wait_prev_scatter(b=b):
                    pltpu.make_async_copy(
                        vals_v.at[b], acc.at[idx_v.at[b]], ssem.at[b]).wait()

                icps.append(pltpu.async_copy(idx_hbm.at[pl.ds(off, ch)],
                                             idx_v.at[b], isem.at[b]))
                vcps.append(pltpu.async_copy(vals_hbm.at[pl.ds(off, ch)],
                                             vals_v.at[b], lsem.at[b]))
            for b in range(_GNB):
                icps[b].wait()
                vcps[b].wait()
                pltpu.async_copy(vals_v.at[b], acc.at[idx_v.at[b]],
                                 ssem.at[b], add=True)
            return carry

        lax.fori_loop(0, ngrp, group, 0, unroll=False)
        for b in range(_GNB):
            pltpu.make_async_copy(vals_v.at[b], acc.at[idx_v.at[b]],
                                  ssem.at[b]).wait()
        plsc.subcore_barrier()
        pltpu.sync_copy(acc.at[pl.ds(r0, ROWS_PER_TILE)],
                        out_hbm.at[c, pl.ds(r0, ROWS_PER_TILE)])

    return scatter


def _sc_scatter_add(vals, idx, zeros):
    return _make_scatter(vals.shape[1])(vals, idx, zeros)


def _ln(v, g, b, eps=1e-5):
    mu = jnp.mean(v, axis=-1, keepdims=True)
    var = jnp.mean((v - mu) ** 2, axis=-1, keepdims=True)
    return (v - mu) / jnp.sqrt(var + eps) * g + b


def _gelu(v):
    return 0.5 * v * (1.0 + lax.erf(v / jnp.sqrt(jnp.float32(2.0))))


def _dot(a, b):
    return jnp.dot(a, b, preferred_element_type=_F32)


def _padw(v, rows):
    return jnp.concatenate([v, jnp.zeros((rows, W - v.shape[1]), _F32)],
                           axis=1)


def _tc_edge_encode(edge_attr, w, b):
    eb = 2000

    def body(ea_ref, w_ref, b_ref, o_ref):
        o_ref[...] = _dot(ea_ref[...], w_ref[...]) + b_ref[...]

    return pl.pallas_call(
        body,
        grid=(E // eb,),
        in_specs=[
            pl.BlockSpec((eb, 16), lambda i: (i, 0)),
            pl.BlockSpec((16, H), lambda i: (0, 0)),
            pl.BlockSpec((1, H), lambda i: (0, 0)),
        ],
        out_specs=pl.BlockSpec((eb, H), lambda i: (i, 0)),
        out_shape=jax.ShapeDtypeStruct((E, H), _F32),
    )(edge_attr, w, b.reshape(1, H))


def _tc_node_pre(x, w_node, wl, bl, wr, br):
    def body(x_ref, wn_ref, wl_ref, bl_ref, wr_ref, br_ref, h_ref, lr_ref):
        h = _dot(x_ref[...], wn_ref[...])
        h_ref[...] = h
        lr_ref[:N] = _dot(h, wl_ref[...]) + bl_ref[...]
        lr_ref[N:] = _dot(h, wr_ref[...]) + br_ref[...]

    return pl.pallas_call(
        body,
        out_shape=[
            jax.ShapeDtypeStruct((N, H), _F32),
            jax.ShapeDtypeStruct((2 * N, H), _F32),
        ],
    )(x, w_node, wl, bl.reshape(1, H), wr, br.reshape(1, H))


def _tc_edge_gat(g_p, ea_p, we4, att4, sones, sexp, sel):
    eb4 = 800
    nb = (E // 4) // eb4

    def body(xls_ref, xrd_ref, ea_ref, we_ref, att_ref, so_ref, se_ref,
             sel_ref, o1_ref, o2_ref):
        xls = xls_ref[...]
        e = xls + xrd_ref[...] + _dot(ea_ref[...], we_ref[...])
        e = jnp.where(e > 0, e, 0.1 * e)
        s4 = _dot(e * att_ref[...], so_ref[...])
        ex4 = jnp.exp(s4)
        exb = _dot(ex4, se_ref[...])
        o1_ref[...] = xls * exb
        o2_ref[...] = _dot(ex4, sel_ref[...])

    return pl.pallas_call(
        body,
        grid=(nb,),
        in_specs=[
            pl.BlockSpec((eb4, W), lambda i: (i, 0)),
            pl.BlockSpec((eb4, W), lambda i: (i + nb, 0)),
            pl.BlockSpec((eb4, W), lambda i: (i, 0)),
            pl.BlockSpec((W, W), lambda i: (0, 0)),
            pl.BlockSpec((1, W), lambda i: (0, 0)),
            pl.BlockSpec((W, 4), lambda i: (0, 0)),
            pl.BlockSpec((4, W), lambda i: (0, 0)),
            pl.BlockSpec((4, W), lambda i: (0, 0)),
        ],
        out_specs=[pl.BlockSpec((eb4, W), lambda i: (i, 0)),
                   pl.BlockSpec((eb4, W), lambda i: (i, 0))],
        out_shape=[jax.ShapeDtypeStruct((E // 4, W), _F32),
                   jax.ShapeDtypeStruct((E // 4, W), _F32)],
    )(g_p, g_p, ea_p, we4, att4, sones, sexp, sel)


def _tc_node_gat(parts, h, ln_g, ln_b, bias, nxt):
    def body(pm_ref, pd_ref, h_ref, g_ref, b_ref, bias_ref, *rest):
        if nxt is None:
            (h_out,) = rest
        else:
            wl_ref, bl_ref, wr_ref, br_ref, h_out, lr_ref = rest
        u = pm_ref[0] + pm_ref[1]
        d = pd_ref[0][:, 0:1] + pd_ref[1][:, 0:1]
        att_out = u / (d + 1e-16) + bias_ref[...]
        h1 = _gelu(_ln(h_ref[...] + att_out, g_ref[...], b_ref[...]))
        h_out[...] = h1
        if nxt is not None:
            lr_ref[:N] = _dot(h1, wl_ref[...]) + bl_ref[...]
            lr_ref[N:] = _dot(h1, wr_ref[...]) + br_ref[...]

    parts_m, parts_d = parts
    out_shape = [jax.ShapeDtypeStruct((N, H), _F32)]
    args = [parts_m, parts_d, h, ln_g.reshape(1, H), ln_b.reshape(1, H),
            bias.reshape(1, H)]
    if nxt is not None:
        wl, bl, wr, br = nxt
        args += [wl, bl.reshape(1, H), wr, br.reshape(1, H)]
        out_shape.append(jax.ShapeDtypeStruct((2 * N, H), _F32))
    return pl.pallas_call(body, out_shape=out_shape)(*args)


def _tc_edge_nn(ea_p, xs_p, w2ext):
    eb4 = 400

    def body(ea_ref, xs_ref, w2_ref, o_ref):
        ident = jax.lax.broadcasted_iota(jnp.int32, (W, W), 0) == \
            jax.lax.broadcasted_iota(jnp.int32, (W, W), 1)
        ident = ident.astype(_F32)
        eat_all = jax.lax.dot_general(ident, ea_ref[...],
                                      (((1,), (1,)), ((), ())),
                                      preferred_element_type=_F32)
        xst_all = jax.lax.dot_general(ident, xs_ref[...],
                                      (((1,), (1,)), ((), ())),
                                      preferred_element_type=_F32)
        msgs = []
        for g in range(4):
            eat = eat_all[H * g:H * (g + 1)]
            xst = xst_all[H * g:H * (g + 1)]
            zt = (eat[:, None, :] * xst[None, :, :]).reshape(H * H, eb4)
            zt = jnp.concatenate([zt, xst], axis=0)
            msgs.append(jax.lax.dot_general(zt, w2_ref[...],
                                            (((0,), (0,)), ((), ())),
                                            preferred_element_type=_F32))
        o_ref[...] = jnp.concatenate(msgs, axis=1)

    return pl.pallas_call(
        body,
        grid=(E // (4 * eb4),),
        in_specs=[
            pl.BlockSpec((eb4, W), lambda i: (i, 0)),
            pl.BlockSpec((eb4, W), lambda i: (i, 0)),
            pl.BlockSpec((H * H + H, H), lambda i: (0, 0)),
        ],
        out_specs=pl.BlockSpec((eb4, W), lambda i: (i, 0)),
        out_shape=jax.ShapeDtypeStruct((E // 4, W), _F32),
    )(ea_p, xs_p, w2ext)


def _tc_node_dec(parts, xin, root, bias, x0, z_prev, ln_g, ln_b):
    def body(p_ref, xin_ref, root_ref, bias_ref, x0_ref, *rest):
        if z_prev is None:
            g_ref, b_ref, t_ref, z_ref = rest
            zp = 0.0
        else:
            zp_ref, g_ref, b_ref, t_ref, z_ref = rest
            zp = zp_ref[...]
        agg = p_ref[0] + p_ref[1]
        hd = zp + agg + _dot(xin_ref[...], root_ref[...]) + bias_ref[...]
        z = x0_ref[...] + hd
        z_ref[...] = z
        t_ref[...] = _gelu(_ln(z, g_ref[...], b_ref[...]))

    args = [parts, xin, root, bias.reshape(1, H), x0]
    if z_prev is not None:
        args.append(z_prev)
    args += [ln_g.reshape(1, H), ln_b.reshape(1, H)]
    return pl.pallas_call(
        body,
        out_shape=[jax.ShapeDtypeStruct((N, H), _F32),
                   jax.ShapeDtypeStruct((N, H), _F32)],
    )(*args)


def _tc_node_final(parts, xin, root, bias, z_prev, ln_g, ln_b, w_out, b_out):
    def body(p_ref, xin_ref, root_ref, bias_ref, zp_ref, g_ref, b_ref,
             wo_ref, bo_ref, o_ref):
        agg = p_ref[0] + p_ref[1]
        hd = (zp_ref[...] + agg + _dot(xin_ref[...], root_ref[...])
              + bias_ref[...])
        t = _gelu(_ln(hd, g_ref[...], b_ref[...]))
        o_ref[...] = _dot(t, wo_ref[...]) + bo_ref[...]

    return pl.pallas_call(
        body,
        out_shape=jax.ShapeDtypeStruct((N, 2), _F32),
    )(parts, xin, root, bias.reshape(1, H), z_prev, ln_g.reshape(1, H),
      ln_b.reshape(1, H), w_out, b_out.reshape(1, 2))


def kernel(x, edge_index, edge_attr, batch, params):
    src = edge_index[0]
    dst = edge_index[1]
    idx_lr = jnp.concatenate([src, dst + N])
    zeros_h = jnp.zeros((ROWS_PER_TILE, H), _F32)

    enc = params['enc']
    dec = params['dec']

    eye4 = jnp.eye(4, dtype=_F32)
    sones = jnp.kron(eye4, jnp.ones((H, 1), _F32))
    sexp = jnp.kron(eye4, jnp.ones((1, H), _F32))
    sel = jnp.kron(eye4, jnp.eye(1, H, dtype=_F32))

    ea = _tc_edge_encode(edge_attr, params['W_edge'], params['b_edge'])
    ea_p = jnp.reshape(ea, (E // 4, W))
    h, lr = _tc_node_pre(x, params['W_node'], enc[0]['Wl'], enc[0]['bl'],
                         enc[0]['Wr'], enc[0]['br'])

    for l in range(len(enc)):
        p = enc[l]
        we4 = jnp.kron(eye4, p['We'])
        att4 = jnp.tile(p['att'], 4).reshape(1, W)
        g = _sc_gather_rows(lr, idx_lr)
        g_p = jnp.reshape(g, (2 * E // 4, W))
        m_p, d_p = _tc_edge_gat(g_p, ea_p, we4, att4, sones, sexp, sel)
        parts_m = _sc_scatter_add(jnp.reshape(m_p, (E, H)), dst, zeros_h)
        parts_d = _sc_scatter_add(jnp.reshape(d_p, (E, H)), dst, zeros_h)
        parts = (parts_m, parts_d)
        if l + 1 < len(enc):
            q = enc[l + 1]
            h, lr = _tc_node_gat(parts, h, p['ln_g'], p['ln_b'], p['bias'],
                                 (q['Wl'], q['bl'], q['Wr'], q['br']))
        else:
            (h,) = _tc_node_gat(parts, h, p['ln_g'], p['ln_b'], p['bias'],
                                None)

    x0 = h
    w2 = [jnp.concatenate([d['nn_W'].reshape(H * H, H),
                           d['nn_b'].reshape(H, H)], axis=0) for d in dec]

    conv_params = [dec[0], dec[0], dec[1], dec[2]]
    conv_w = [w2[0], w2[0], w2[1], w2[2]]
    ln_next = [dec[0], dec[1], dec[2]]

    xin, z = x0, None
    for j in range(4):
        cp = conv_params[j]
        xs = _sc_gather_rows(xin, src)
        msg_p = _tc_edge_nn(ea_p, jnp.reshape(xs, (E // 4, W)), conv_w[j])
        parts = _sc_scatter_add(jnp.reshape(msg_p, (E, H)), dst, zeros_h)
        if j < 3:
            nl = ln_next[j]
            t, z = _tc_node_dec(parts, xin, cp['root'], cp['bias'],
                                x0, z, nl['ln_g'], nl['ln_b'])
            xin = t
        else:
            out = _tc_node_final(parts, xin, cp['root'], cp['bias'], z,
                                 dec[0]['ln_g'], dec[0]['ln_b'],
                                 params['W_out'], params['b_out'])
    return out


# speedup vs baseline: 7.1120x; 1.0424x over previous
import functools

import jax
import jax.numpy as jnp
from jax import lax
from jax.experimental import pallas as pl
from jax.experimental.pallas import tpu as pltpu
from jax.experimental.pallas import tpu_sc as plsc

N = 10000
E = 160000
H = 32
W = 128
NW = 32
ROWS_PER_TILE = N // 16

_GCH = 40
_GNB = 5

_F32 = jnp.float32


def _sc_mesh():
    return plsc.VectorSubcoreMesh(core_axis_name="c", subcore_axis_name="s")


@functools.lru_cache(maxsize=None)
def _make_gather(n_rows, w):
    ch = 40 if w == W else 200
    wpw = n_rows // NW
    nch = wpw // ch
    ngrp = nch // _GNB

    @functools.partial(
        pl.kernel,
        out_type=jax.ShapeDtypeStruct((n_rows, w), _F32),
        mesh=_sc_mesh(),
        scratch_types=[
            pltpu.VMEM((wpw,), jnp.int32),
            pltpu.VMEM((_GNB, ch, w), _F32),
            pltpu.SemaphoreType.DMA,
            pltpu.SemaphoreType.DMA((_GNB,)),
            pltpu.SemaphoreType.DMA((_GNB,)),
        ],
        compiler_params=pltpu.CompilerParams(use_tc_tiling_on_sc=False),
    )
    def gather(table_hbm, idx_hbm, out_hbm, idx_all, rows, isem, gsem, wsem):
        wid = lax.axis_index("s") * 2 + lax.axis_index("c")
        base = wid * wpw
        pltpu.async_copy(idx_hbm.at[pl.ds(base, wpw)], idx_all, isem).wait()

        def group(gi, carry):
            cps = []
            for b in range(_GNB):
                off = (gi * _GNB + b) * ch

                @pl.when(gi > 0)
                def _wait_prev_write(b=b):
                    pltpu.make_async_copy(
                        rows.at[b], out_hbm.at[pl.ds(base, ch)],
                        wsem.at[b]).wait()

                cps.append(pltpu.async_copy(
                    table_hbm.at[idx_all.at[pl.ds(off, ch)]],
                    rows.at[b], gsem.at[b]))
            for b in range(_GNB):
                off = (gi * _GNB + b) * ch
                cps[b].wait()
                pltpu.async_copy(rows.at[b],
                                 out_hbm.at[pl.ds(base + off, ch)],
                                 wsem.at[b])
            return carry

        lax.fori_loop(0, ngrp, group, 0, unroll=False)
        for b in range(_GNB):
            pltpu.make_async_copy(rows.at[b], out_hbm.at[pl.ds(base, ch)],
                                  wsem.at[b]).wait()

    return gather


def _sc_gather_rows(table, idx):
    return _make_gather(idx.shape[0], table.shape[1])(table, idx)


@functools.lru_cache(maxsize=None)
def _make_scatter(w):
    ch = 40 if w == W else 200
    wpw = E // NW
    nch = wpw // ch
    ngrp = nch // _GNB

    @functools.partial(
        pl.kernel,
        out_type=jax.ShapeDtypeStruct((2, N, w), _F32),
        mesh=_sc_mesh(),
        scratch_types=[
            pltpu.VMEM((_GNB, ch), jnp.int32),
            pltpu.VMEM((_GNB, ch, w), _F32),
            pltpu.VMEM_SHARED((N, w), _F32),
            pltpu.SemaphoreType.DMA((_GNB,)),
            pltpu.SemaphoreType.DMA((_GNB,)),
            pltpu.SemaphoreType.DMA((_GNB,)),
        ],
        compiler_params=pltpu.CompilerParams(use_tc_tiling_on_sc=False),
    )
    def scatter(vals_hbm, idx_hbm, zeros_hbm, out_hbm, idx_v, vals_v, acc,
                isem, lsem, ssem):
        c = lax.axis_index("c")
        s = lax.axis_index("s")
        wid = s * 2 + c
        r0 = s * ROWS_PER_TILE
        pltpu.sync_copy(zeros_hbm, acc.at[pl.ds(r0, ROWS_PER_TILE)])
        plsc.subcore_barrier()

        base = wid * wpw

        def group(gi, carry):
            icps, vcps = [], []
            for b in range(_GNB):
                off = base + (gi * _GNB + b) * ch

                @pl.when(gi > 0)
                def _wait_prev_scatter(b=b):
                    pltpu.make_async_copy(
                        vals_v.at[b], acc.at[idx_v.at[b]], ssem.at[b]).wait()

                icps.append(pltpu.async_copy(idx_hbm.at[pl.ds(off, ch)],
                                             idx_v.at[b], isem.at[b]))
                vcps.append(pltpu.async_copy(vals_hbm.at[pl.ds(off, ch)],
                                             vals_v.at[b], lsem.at[b]))
            for b in range(_GNB):
                icps[b].wait()
                vcps[b].wait()
                pltpu.async_copy(vals_v.at[b], acc.at[idx_v.at[b]],
                                 ssem.at[b], add=True)
            return carry

        lax.fori_loop(0, ngrp, group, 0, unroll=False)
        for b in range(_GNB):
            pltpu.make_async_copy(vals_v.at[b], acc.at[idx_v.at[b]],
                                  ssem.at[b]).wait()
        plsc.subcore_barrier()
        pltpu.sync_copy(acc.at[pl.ds(r0, ROWS_PER_TILE)],
                        out_hbm.at[c, pl.ds(r0, ROWS_PER_TILE)])

    return scatter


def _sc_scatter_add(vals, idx, zeros):
    return _make_scatter(vals.shape[1])(vals, idx, zeros)


def _ln(v, g, b, eps=1e-5):
    mu = jnp.mean(v, axis=-1, keepdims=True)
    var = jnp.mean((v - mu) ** 2, axis=-1, keepdims=True)
    return (v - mu) / jnp.sqrt(var + eps) * g + b


def _gelu(v):
    return 0.5 * v * (1.0 + lax.erf(v / jnp.sqrt(jnp.float32(2.0))))


def _dot(a, b):
    return jnp.dot(a, b, preferred_element_type=_F32)


def _padw(v, rows):
    return jnp.concatenate([v, jnp.zeros((rows, W - v.shape[1]), _F32)],
                           axis=1)


def _tc_edge_encode(edge_attr, w, b):
    eb = 2000

    def body(ea_ref, w_ref, b_ref, o_ref):
        o_ref[...] = _dot(ea_ref[...], w_ref[...]) + b_ref[...]

    return pl.pallas_call(
        body,
        grid=(E // eb,),
        in_specs=[
            pl.BlockSpec((eb, 16), lambda i: (i, 0)),
            pl.BlockSpec((16, H), lambda i: (0, 0)),
            pl.BlockSpec((1, H), lambda i: (0, 0)),
        ],
        out_specs=pl.BlockSpec((eb, H), lambda i: (i, 0)),
        out_shape=jax.ShapeDtypeStruct((E, H), _F32),
    )(edge_attr, w, b.reshape(1, H))


def _tc_node_pre(x, w_node, wl, bl, wr, br):
    def body(x_ref, wn_ref, wl_ref, bl_ref, wr_ref, br_ref, h_ref, lr_ref):
        h = _dot(x_ref[...], wn_ref[...])
        h_ref[...] = h
        lr_ref[:N] = _dot(h, wl_ref[...]) + bl_ref[...]
        lr_ref[N:] = _dot(h, wr_ref[...]) + br_ref[...]

    return pl.pallas_call(
        body,
        out_shape=[
            jax.ShapeDtypeStruct((N, H), _F32),
            jax.ShapeDtypeStruct((2 * N, H), _F32),
        ],
    )(x, w_node, wl, bl.reshape(1, H), wr, br.reshape(1, H))


def _tc_edge_gat(g_p, ea_p, we4, att4, sones, sexp, sel):
    eb4 = 800
    nb = (E // 4) // eb4

    def body(xls_ref, xrd_ref, ea_ref, we_ref, att_ref, so_ref, se_ref,
             sel_ref, o1_ref, o2_ref):
        xls = xls_ref[...]
        e = xls + xrd_ref[...] + _dot(ea_ref[...], we_ref[...])
        e = jnp.where(e > 0, e, 0.1 * e)
        s4 = _dot(e * att_ref[...], so_ref[...])
        ex4 = jnp.exp(s4)
        exb = _dot(ex4, se_ref[...])
        o1_ref[...] = xls * exb
        o2_ref[...] = _dot(ex4, sel_ref[...])

    return pl.pallas_call(
        body,
        grid=(nb,),
        in_specs=[
            pl.BlockSpec((eb4, W), lambda i: (i, 0)),
            pl.BlockSpec((eb4, W), lambda i: (i + nb, 0)),
            pl.BlockSpec((eb4, W), lambda i: (i, 0)),
            pl.BlockSpec((W, W), lambda i: (0, 0)),
            pl.BlockSpec((1, W), lambda i: (0, 0)),
            pl.BlockSpec((W, 4), lambda i: (0, 0)),
            pl.BlockSpec((4, W), lambda i: (0, 0)),
            pl.BlockSpec((4, W), lambda i: (0, 0)),
        ],
        out_specs=[pl.BlockSpec((eb4, W), lambda i: (i, 0)),
                   pl.BlockSpec((eb4, W), lambda i: (i, 0))],
        out_shape=[jax.ShapeDtypeStruct((E // 4, W), _F32),
                   jax.ShapeDtypeStruct((E // 4, W), _F32)],
    )(g_p, g_p, ea_p, we4, att4, sones, sexp, sel)


def _tc_node_gat(parts, h, ln_g, ln_b, bias, nxt):
    def body(pm_ref, pd_ref, h_ref, g_ref, b_ref, bias_ref, *rest):
        if nxt is None:
            (h_out,) = rest
        else:
            wl_ref, bl_ref, wr_ref, br_ref, h_out, lr_ref = rest
        u = pm_ref[0] + pm_ref[1]
        d = pd_ref[0][:, 0:1] + pd_ref[1][:, 0:1]
        att_out = u / (d + 1e-16) + bias_ref[...]
        h1 = _gelu(_ln(h_ref[...] + att_out, g_ref[...], b_ref[...]))
        h_out[...] = h1
        if nxt is not None:
            lr_ref[:N] = _dot(h1, wl_ref[...]) + bl_ref[...]
            lr_ref[N:] = _dot(h1, wr_ref[...]) + br_ref[...]

    parts_m, parts_d = parts
    out_shape = [jax.ShapeDtypeStruct((N, H), _F32)]
    args = [parts_m, parts_d, h, ln_g.reshape(1, H), ln_b.reshape(1, H),
            bias.reshape(1, H)]
    if nxt is not None:
        wl, bl, wr, br = nxt
        args += [wl, bl.reshape(1, H), wr, br.reshape(1, H)]
        out_shape.append(jax.ShapeDtypeStruct((2 * N, H), _F32))
    return pl.pallas_call(body, out_shape=out_shape)(*args)


def _tc_edge_nn(ea_p, xs_p, w2ext):
    eb4 = 400

    def body(ea_ref, xs_ref, w2_ref, o_ref):
        ident = jax.lax.broadcasted_iota(jnp.int32, (W, W), 0) == \
            jax.lax.broadcasted_iota(jnp.int32, (W, W), 1)
        ident = ident.astype(_F32)
        eat_all = jax.lax.dot_general(ident, ea_ref[...],
                                      (((1,), (1,)), ((), ())),
                                      preferred_element_type=_F32)
        xst_all = jax.lax.dot_general(ident, xs_ref[...],
                                      (((1,), (1,)), ((), ())),
                                      preferred_element_type=_F32)
        w2b = w2_ref[...].astype(jnp.bfloat16)
        msgs = []
        for g in range(4):
            eat = eat_all[H * g:H * (g + 1)]
            xst = xst_all[H * g:H * (g + 1)]
            zt = (eat[:, None, :] * xst[None, :, :]).reshape(H * H, eb4)
            zt = jnp.concatenate([zt, xst], axis=0)
            msgs.append(jax.lax.dot_general(zt.astype(jnp.bfloat16), w2b,
                                            (((0,), (0,)), ((), ())),
                                            preferred_element_type=_F32))
        o_ref[...] = jnp.concatenate(msgs, axis=1)

    return pl.pallas_call(
        body,
        grid=(E // (4 * eb4),),
        in_specs=[
            pl.BlockSpec((eb4, W), lambda i: (i, 0)),
            pl.BlockSpec((eb4, W), lambda i: (i, 0)),
            pl.BlockSpec((H * H + H, H), lambda i: (0, 0)),
        ],
        out_specs=pl.BlockSpec((eb4, W), lambda i: (i, 0)),
        out_shape=jax.ShapeDtypeStruct((E // 4, W), _F32),
    )(ea_p, xs_p, w2ext)


def _tc_node_dec(parts, xin, root, bias, x0, z_prev, ln_g, ln_b):
    def body(p_ref, xin_ref, root_ref, bias_ref, x0_ref, *rest):
        if z_prev is None:
            g_ref, b_ref, t_ref, z_ref = rest
            zp = 0.0
        else:
            zp_ref, g_ref, b_ref, t_ref, z_ref = rest
            zp = zp_ref[...]
        agg = p_ref[0] + p_ref[1]
        hd = zp + agg + _dot(xin_ref[...], root_ref[...]) + bias_ref[...]
        z = x0_ref[...] + hd
        z_ref[...] = z
        t_ref[...] = _gelu(_ln(z, g_ref[...], b_ref[...]))

    args = [parts, xin, root, bias.reshape(1, H), x0]
    if z_prev is not None:
        args.append(z_prev)
    args += [ln_g.reshape(1, H), ln_b.reshape(1, H)]
    return pl.pallas_call(
        body,
        out_shape=[jax.ShapeDtypeStruct((N, H), _F32),
                   jax.ShapeDtypeStruct((N, H), _F32)],
    )(*args)


def _tc_node_final(parts, xin, root, bias, z_prev, ln_g, ln_b, w_out, b_out):
    def body(p_ref, xin_ref, root_ref, bias_ref, zp_ref, g_ref, b_ref,
             wo_ref, bo_ref, o_ref):
        agg = p_ref[0] + p_ref[1]
        hd = (zp_ref[...] + agg + _dot(xin_ref[...], root_ref[...])
              + bias_ref[...])
        t = _gelu(_ln(hd, g_ref[...], b_ref[...]))
        o_ref[...] = _dot(t, wo_ref[...]) + bo_ref[...]

    return pl.pallas_call(
        body,
        out_shape=jax.ShapeDtypeStruct((N, 2), _F32),
    )(parts, xin, root, bias.reshape(1, H), z_prev, ln_g.reshape(1, H),
      ln_b.reshape(1, H), w_out, b_out.reshape(1, 2))


def kernel(x, edge_index, edge_attr, batch, params):
    src = edge_index[0]
    dst = edge_index[1]
    idx_lr = jnp.concatenate([src, dst + N])
    zeros_h = jnp.zeros((ROWS_PER_TILE, H), _F32)

    enc = params['enc']
    dec = params['dec']

    eye4 = jnp.eye(4, dtype=_F32)
    sones = jnp.kron(eye4, jnp.ones((H, 1), _F32))
    sexp = jnp.kron(eye4, jnp.ones((1, H), _F32))
    sel = jnp.kron(eye4, jnp.eye(1, H, dtype=_F32))

    ea = _tc_edge_encode(edge_attr, params['W_edge'], params['b_edge'])
    ea_p = jnp.reshape(ea, (E // 4, W))
    h, lr = _tc_node_pre(x, params['W_node'], enc[0]['Wl'], enc[0]['bl'],
                         enc[0]['Wr'], enc[0]['br'])

    for l in range(len(enc)):
        p = enc[l]
        we4 = jnp.kron(eye4, p['We'])
        att4 = jnp.tile(p['att'], 4).reshape(1, W)
        g = _sc_gather_rows(lr, idx_lr)
        g_p = jnp.reshape(g, (2 * E // 4, W))
        m_p, d_p = _tc_edge_gat(g_p, ea_p, we4, att4, sones, sexp, sel)
        parts_m = _sc_scatter_add(jnp.reshape(m_p, (E, H)), dst, zeros_h)
        parts_d = _sc_scatter_add(jnp.reshape(d_p, (E, H)), dst, zeros_h)
        parts = (parts_m, parts_d)
        if l + 1 < len(enc):
            q = enc[l + 1]
            h, lr = _tc_node_gat(parts, h, p['ln_g'], p['ln_b'], p['bias'],
                                 (q['Wl'], q['bl'], q['Wr'], q['br']))
        else:
            (h,) = _tc_node_gat(parts, h, p['ln_g'], p['ln_b'], p['bias'],
                                None)

    x0 = h
    w2 = [jnp.concatenate([d['nn_W'].reshape(H * H, H),
                           d['nn_b'].reshape(H, H)], axis=0) for d in dec]

    conv_params = [dec[0], dec[0], dec[1], dec[2]]
    conv_w = [w2[0], w2[0], w2[1], w2[2]]
    ln_next = [dec[0], dec[1], dec[2]]

    xin, z = x0, None
    for j in range(4):
        cp = conv_params[j]
        xs = _sc_gather_rows(xin, src)
        msg_p = _tc_edge_nn(ea_p, jnp.reshape(xs, (E // 4, W)), conv_w[j])
        parts = _sc_scatter_add(jnp.reshape(msg_p, (E, H)), dst, zeros_h)
        if j < 3:
            nl = ln_next[j]
            t, z = _tc_node_dec(parts, xin, cp['root'], cp['bias'],
                                x0, z, nl['ln_g'], nl['ln_b'])
            xin = t
        else:
            out = _tc_node_final(parts, xin, cp['root'], cp['bias'], z,
                                 dec[0]['ln_g'], dec[0]['ln_b'],
                                 params['W_out'], params['b_out'])
    return out


# speedup vs baseline: 7.2312x; 1.0168x over previous
import functools

import jax
import jax.numpy as jnp
from jax import lax
from jax.experimental import pallas as pl
from jax.experimental.pallas import tpu as pltpu
from jax.experimental.pallas import tpu_sc as plsc

N = 10000
E = 160000
H = 32
W = 128
NW = 32
ROWS_PER_TILE = N // 16

_GCH = 40
_GNB = 5

_F32 = jnp.float32


def _sc_mesh():
    return plsc.VectorSubcoreMesh(core_axis_name="c", subcore_axis_name="s")


@functools.lru_cache(maxsize=None)
def _make_gather(n_rows, w):
    ch = 40 if w == W else 200
    wpw = n_rows // NW
    nch = wpw // ch
    ngrp = nch // _GNB

    @functools.partial(
        pl.kernel,
        out_type=jax.ShapeDtypeStruct((n_rows, w), _F32),
        mesh=_sc_mesh(),
        scratch_types=[
            pltpu.VMEM((wpw,), jnp.int32),
            pltpu.VMEM((_GNB, ch, w), _F32),
            pltpu.SemaphoreType.DMA,
            pltpu.SemaphoreType.DMA((_GNB,)),
            pltpu.SemaphoreType.DMA((_GNB,)),
        ],
        compiler_params=pltpu.CompilerParams(use_tc_tiling_on_sc=False),
    )
    def gather(table_hbm, idx_hbm, out_hbm, idx_all, rows, isem, gsem, wsem):
        wid = lax.axis_index("s") * 2 + lax.axis_index("c")
        base = wid * wpw
        pltpu.async_copy(idx_hbm.at[pl.ds(base, wpw)], idx_all, isem).wait()

        def group(gi, carry):
            cps = []
            for b in range(_GNB):
                off = (gi * _GNB + b) * ch

                @pl.when(gi > 0)
                def _wait_prev_write(b=b):
                    pltpu.make_async_copy(
                        rows.at[b], out_hbm.at[pl.ds(base, ch)],
                        wsem.at[b]).wait()

                cps.append(pltpu.async_copy(
                    table_hbm.at[idx_all.at[pl.ds(off, ch)]],
                    rows.at[b], gsem.at[b]))
            for b in range(_GNB):
                off = (gi * _GNB + b) * ch
                cps[b].wait()
                pltpu.async_copy(rows.at[b],
                                 out_hbm.at[pl.ds(base + off, ch)],
                                 wsem.at[b])
            return carry

        lax.fori_loop(0, ngrp, group, 0, unroll=False)
        for b in range(_GNB):
            pltpu.make_async_copy(rows.at[b], out_hbm.at[pl.ds(base, ch)],
                                  wsem.at[b]).wait()

    return gather


def _sc_gather_rows(table, idx):
    return _make_gather(idx.shape[0], table.shape[1])(table, idx)


@functools.lru_cache(maxsize=None)
def _make_scatter(w):
    ch = 40 if w == W else 200
    wpw = E // NW
    nch = wpw // ch
    ngrp = nch // _GNB

    @functools.partial(
        pl.kernel,
        out_type=jax.ShapeDtypeStruct((2, N, w), _F32),
        mesh=_sc_mesh(),
        scratch_types=[
            pltpu.VMEM((_GNB, ch), jnp.int32),
            pltpu.VMEM((_GNB, ch, w), _F32),
            pltpu.VMEM_SHARED((N, w), _F32),
            pltpu.SemaphoreType.DMA((_GNB,)),
            pltpu.SemaphoreType.DMA((_GNB,)),
            pltpu.SemaphoreType.DMA((_GNB,)),
        ],
        compiler_params=pltpu.CompilerParams(use_tc_tiling_on_sc=False),
    )
    def scatter(vals_hbm, idx_hbm, zeros_hbm, out_hbm, idx_v, vals_v, acc,
                isem, lsem, ssem):
        c = lax.axis_index("c")
        s = lax.axis_index("s")
        wid = s * 2 + c
        r0 = s * ROWS_PER_TILE
        pltpu.sync_copy(zeros_hbm, acc.at[pl.ds(r0, ROWS_PER_TILE)])
        plsc.subcore_barrier()

        base = wid * wpw

        def group(gi, carry):
            icps, vcps = [], []
            for b in range(_GNB):
                off = base + (gi * _GNB + b) * ch

                @pl.when(gi > 0)
                def _wait_prev_scatter(b=b):
                    pltpu.make_async_copy(
                        vals_v.at[b], acc.at[idx_v.at[b]], ssem.at[b]).wait()

                icps.append(pltpu.async_copy(idx_hbm.at[pl.ds(off, ch)],
                                             idx_v.at[b], isem.at[b]))
                vcps.append(pltpu.async_copy(vals_hbm.at[pl.ds(off, ch)],
                                             vals_v.at[b], lsem.at[b]))
            for b in range(_GNB):
                icps[b].wait()
                vcps[b].wait()
                pltpu.async_copy(vals_v.at[b], acc.at[idx_v.at[b]],
                                 ssem.at[b], add=True)
            return carry

        lax.fori_loop(0, ngrp, group, 0, unroll=False)
        for b in range(_GNB):
            pltpu.make_async_copy(vals_v.at[b], acc.at[idx_v.at[b]],
                                  ssem.at[b]).wait()
        plsc.subcore_barrier()
        pltpu.sync_copy(acc.at[pl.ds(r0, ROWS_PER_TILE)],
                        out_hbm.at[c, pl.ds(r0, ROWS_PER_TILE)])

    return scatter


def _sc_scatter_add(vals, idx, zeros):
    return _make_scatter(vals.shape[1])(vals, idx, zeros)


def _ln(v, g, b, eps=1e-5):
    mu = jnp.mean(v, axis=-1, keepdims=True)
    var = jnp.mean((v - mu) ** 2, axis=-1, keepdims=True)
    return (v - mu) / jnp.sqrt(var + eps) * g + b


def _gelu(v):
    return 0.5 * v * (1.0 + lax.erf(v / jnp.sqrt(jnp.float32(2.0))))


def _dot(a, b):
    return jnp.dot(a, b, preferred_element_type=_F32)


def _padw(v, rows):
    return jnp.concatenate([v, jnp.zeros((rows, W - v.shape[1]), _F32)],
                           axis=1)


def _tc_edge_encode(edge_attr, w, b):
    eb = 2000

    def body(ea_ref, w_ref, b_ref, o_ref):
        o_ref[...] = _dot(ea_ref[...], w_ref[...]) + b_ref[...]

    return pl.pallas_call(
        body,
        grid=(E // eb,),
        in_specs=[
            pl.BlockSpec((eb, 16), lambda i: (i, 0)),
            pl.BlockSpec((16, H), lambda i: (0, 0)),
            pl.BlockSpec((1, H), lambda i: (0, 0)),
        ],
        out_specs=pl.BlockSpec((eb, H), lambda i: (i, 0)),
        out_shape=jax.ShapeDtypeStruct((E, H), _F32),
    )(edge_attr, w, b.reshape(1, H))


def _tc_node_pre(x, w_node, wl, bl, wr, br):
    def body(x_ref, wn_ref, wl_ref, bl_ref, wr_ref, br_ref, h_ref, lr_ref):
        h = _dot(x_ref[...], wn_ref[...])
        h_ref[...] = h
        lr_ref[:N] = _dot(h, wl_ref[...]) + bl_ref[...]
        lr_ref[N:] = _dot(h, wr_ref[...]) + br_ref[...]

    return pl.pallas_call(
        body,
        out_shape=[
            jax.ShapeDtypeStruct((N, H), _F32),
            jax.ShapeDtypeStruct((2 * N, H), _F32),
        ],
    )(x, w_node, wl, bl.reshape(1, H), wr, br.reshape(1, H))


def _tc_edge_gat(g_p, ea_p, we4, att4, sones, sexp, sel):
    eb4 = 800
    nb = (E // 4) // eb4

    def body(xls_ref, xrd_ref, ea_ref, we_ref, att_ref, so_ref, se_ref,
             sel_ref, o1_ref, o2_ref):
        xls = xls_ref[...]
        e = xls + xrd_ref[...] + _dot(ea_ref[...], we_ref[...])
        e = jnp.where(e > 0, e, 0.1 * e)
        s4 = _dot(e * att_ref[...], so_ref[...])
        ex4 = jnp.exp(s4)
        exb = _dot(ex4, se_ref[...])
        o1_ref[...] = xls * exb
        o2_ref[...] = _dot(ex4, sel_ref[...])

    return pl.pallas_call(
        body,
        grid=(nb,),
        in_specs=[
            pl.BlockSpec((eb4, W), lambda i: (i, 0)),
            pl.BlockSpec((eb4, W), lambda i: (i + nb, 0)),
            pl.BlockSpec((eb4, W), lambda i: (i, 0)),
            pl.BlockSpec((W, W), lambda i: (0, 0)),
            pl.BlockSpec((1, W), lambda i: (0, 0)),
            pl.BlockSpec((W, 4), lambda i: (0, 0)),
            pl.BlockSpec((4, W), lambda i: (0, 0)),
            pl.BlockSpec((4, W), lambda i: (0, 0)),
        ],
        out_specs=[pl.BlockSpec((eb4, W), lambda i: (i, 0)),
                   pl.BlockSpec((eb4, W), lambda i: (i, 0))],
        out_shape=[jax.ShapeDtypeStruct((E // 4, W), _F32),
                   jax.ShapeDtypeStruct((E // 4, W), _F32)],
    )(g_p, g_p, ea_p, we4, att4, sones, sexp, sel)


def _tc_node_gat(parts, h, ln_g, ln_b, bias, nxt):
    def body(pm_ref, pd_ref, h_ref, g_ref, b_ref, bias_ref, *rest):
        if nxt is None:
            (h_out,) = rest
        else:
            wl_ref, bl_ref, wr_ref, br_ref, h_out, lr_ref = rest
        u = pm_ref[0] + pm_ref[1]
        d = pd_ref[0][:, 0:1] + pd_ref[1][:, 0:1]
        att_out = u / (d + 1e-16) + bias_ref[...]
        h1 = _gelu(_ln(h_ref[...] + att_out, g_ref[...], b_ref[...]))
        h_out[...] = h1
        if nxt is not None:
            lr_ref[:N] = _dot(h1, wl_ref[...]) + bl_ref[...]
            lr_ref[N:] = _dot(h1, wr_ref[...]) + br_ref[...]

    parts_m, parts_d = parts
    out_shape = [jax.ShapeDtypeStruct((N, H), _F32)]
    args = [parts_m, parts_d, h, ln_g.reshape(1, H), ln_b.reshape(1, H),
            bias.reshape(1, H)]
    if nxt is not None:
        wl, bl, wr, br = nxt
        args += [wl, bl.reshape(1, H), wr, br.reshape(1, H)]
        out_shape.append(jax.ShapeDtypeStruct((2 * N, H), _F32))
    return pl.pallas_call(body, out_shape=out_shape)(*args)


def _tc_edge_nn(ea_p, xs_p, w2ext):
    eb4 = 400

    def body(ea_ref, xs_ref, w2_ref, o_ref):
        ident = jax.lax.broadcasted_iota(jnp.int32, (W, W), 0) == \
            jax.lax.broadcasted_iota(jnp.int32, (W, W), 1)
        ident = ident.astype(jnp.bfloat16)
        eat_all = jax.lax.dot_general(
            ident, ea_ref[...].astype(jnp.bfloat16),
            (((1,), (1,)), ((), ())),
            preferred_element_type=_F32).astype(jnp.bfloat16)
        xst_all = jax.lax.dot_general(
            ident, xs_ref[...].astype(jnp.bfloat16),
            (((1,), (1,)), ((), ())),
            preferred_element_type=_F32).astype(jnp.bfloat16)
        w2b = w2_ref[...].astype(jnp.bfloat16)
        msgs = []
        for g in range(4):
            eat = eat_all[H * g:H * (g + 1)]
            xst = xst_all[H * g:H * (g + 1)]
            zt = (eat[:, None, :] * xst[None, :, :]).reshape(H * H, eb4)
            zt = jnp.concatenate([zt, xst], axis=0)
            msgs.append(jax.lax.dot_general(zt, w2b,
                                            (((0,), (0,)), ((), ())),
                                            preferred_element_type=_F32))
        o_ref[...] = jnp.concatenate(msgs, axis=1)

    return pl.pallas_call(
        body,
        grid=(E // (4 * eb4),),
        in_specs=[
            pl.BlockSpec((eb4, W), lambda i: (i, 0)),
            pl.BlockSpec((eb4, W), lambda i: (i, 0)),
            pl.BlockSpec((H * H + H, H), lambda i: (0, 0)),
        ],
        out_specs=pl.BlockSpec((eb4, W), lambda i: (i, 0)),
        out_shape=jax.ShapeDtypeStruct((E // 4, W), _F32),
    )(ea_p, xs_p, w2ext)


def _tc_node_dec(parts, xin, root, bias, x0, z_prev, ln_g, ln_b):
    def body(p_ref, xin_ref, root_ref, bias_ref, x0_ref, *rest):
        if z_prev is None:
            g_ref, b_ref, t_ref, z_ref = rest
            zp = 0.0
        else:
            zp_ref, g_ref, b_ref, t_ref, z_ref = rest
            zp = zp_ref[...]
        agg = p_ref[0] + p_ref[1]
        hd = zp + agg + _dot(xin_ref[...], root_ref[...]) + bias_ref[...]
        z = x0_ref[...] + hd
        z_ref[...] = z
        t_ref[...] = _gelu(_ln(z, g_ref[...], b_ref[...]))

    args = [parts, xin, root, bias.reshape(1, H), x0]
    if z_prev is not None:
        args.append(z_prev)
    args += [ln_g.reshape(1, H), ln_b.reshape(1, H)]
    return pl.pallas_call(
        body,
        out_shape=[jax.ShapeDtypeStruct((N, H), _F32),
                   jax.ShapeDtypeStruct((N, H), _F32)],
    )(*args)


def _tc_node_final(parts, xin, root, bias, z_prev, ln_g, ln_b, w_out, b_out):
    def body(p_ref, xin_ref, root_ref, bias_ref, zp_ref, g_ref, b_ref,
             wo_ref, bo_ref, o_ref):
        agg = p_ref[0] + p_ref[1]
        hd = (zp_ref[...] + agg + _dot(xin_ref[...], root_ref[...])
              + bias_ref[...])
        t = _gelu(_ln(hd, g_ref[...], b_ref[...]))
        o_ref[...] = _dot(t, wo_ref[...]) + bo_ref[...]

    return pl.pallas_call(
        body,
        out_shape=jax.ShapeDtypeStruct((N, 2), _F32),
    )(parts, xin, root, bias.reshape(1, H), z_prev, ln_g.reshape(1, H),
      ln_b.reshape(1, H), w_out, b_out.reshape(1, 2))


def kernel(x, edge_index, edge_attr, batch, params):
    src = edge_index[0]
    dst = edge_index[1]
    idx_lr = jnp.concatenate([src, dst + N])
    zeros_h = jnp.zeros((ROWS_PER_TILE, H), _F32)

    enc = params['enc']
    dec = params['dec']

    eye4 = jnp.eye(4, dtype=_F32)
    sones = jnp.kron(eye4, jnp.ones((H, 1), _F32))
    sexp = jnp.kron(eye4, jnp.ones((1, H), _F32))
    sel = jnp.kron(eye4, jnp.eye(1, H, dtype=_F32))

    ea = _tc_edge_encode(edge_attr, params['W_edge'], params['b_edge'])
    ea_p = jnp.reshape(ea, (E // 4, W))
    h, lr = _tc_node_pre(x, params['W_node'], enc[0]['Wl'], enc[0]['bl'],
                         enc[0]['Wr'], enc[0]['br'])

    for l in range(len(enc)):
        p = enc[l]
        we4 = jnp.kron(eye4, p['We'])
        att4 = jnp.tile(p['att'], 4).reshape(1, W)
        g = _sc_gather_rows(lr, idx_lr)
        g_p = jnp.reshape(g, (2 * E // 4, W))
        m_p, d_p = _tc_edge_gat(g_p, ea_p, we4, att4, sones, sexp, sel)
        parts_m = _sc_scatter_add(jnp.reshape(m_p, (E, H)), dst, zeros_h)
        parts_d = _sc_scatter_add(jnp.reshape(d_p, (E, H)), dst, zeros_h)
        parts = (parts_m, parts_d)
        if l + 1 < len(enc):
            q = enc[l + 1]
            h, lr = _tc_node_gat(parts, h, p['ln_g'], p['ln_b'], p['bias'],
                                 (q['Wl'], q['bl'], q['Wr'], q['br']))
        else:
            (h,) = _tc_node_gat(parts, h, p['ln_g'], p['ln_b'], p['bias'],
                                None)

    x0 = h
    w2 = [jnp.concatenate([d['nn_W'].reshape(H * H, H),
                           d['nn_b'].reshape(H, H)], axis=0) for d in dec]

    conv_params = [dec[0], dec[0], dec[1], dec[2]]
    conv_w = [w2[0], w2[0], w2[1], w2[2]]
    ln_next = [dec[0], dec[1], dec[2]]

    xin, z = x0, None
    for j in range(4):
        cp = conv_params[j]
        xs = _sc_gather_rows(xin, src)
        msg_p = _tc_edge_nn(ea_p, jnp.reshape(xs, (E // 4, W)), conv_w[j])
        parts = _sc_scatter_add(jnp.reshape(msg_p, (E, H)), dst, zeros_h)
        if j < 3:
            nl = ln_next[j]
            t, z = _tc_node_dec(parts, xin, cp['root'], cp['bias'],
                                x0, z, nl['ln_g'], nl['ln_b'])
            xin = t
        else:
            out = _tc_node_final(parts, xin, cp['root'], cp['bias'], z,
                                 dec[0]['ln_g'], dec[0]['ln_b'],
                                 params['W_out'], params['b_out'])
    return out
